# baseline (speedup 1.0000x reference)
import jax
import jax.numpy as jnp
import numpy as np
from jax import lax
from jax.experimental import pallas as pl
from jax.experimental.pallas import tpu as pltpu
from jax.experimental.pallas import tpu_sc as plsc

_B, _N, _C, _LC = 4, 20000, 20, 10
_TH = 0.05
_MAXD = 300
_KCAP = 304
_NG = _N // 16
_NSG = (_NG + 15) // 16
_L1P = _NSG * 16
_L2P = 80
_NT = _B * _C
_NW = 32
_NEG = float("-inf")
_LCROW = 128

_mesh = plsc.VectorSubcoreMesh(
    core_axis_name="c", subcore_axis_name="s", num_cores=2, num_subcores=16
)
_params = pltpu.CompilerParams(needs_layout_passes=False)


_SCPAD = _L1P * 16
_L1PAD = _L2P * 16
_BIG = 3.0e38


def _iota():
    return lax.iota(jnp.int32, 16)


def _minwhere(mask):
    return plsc.all_reduce_ffs(mask)[0]


def _bfull(x, dtype=None):
    v = jnp.full((16,), x)
    return v if dtype is None else v.astype(dtype)


def _al8(x):
    return pl.multiple_of(x, 8)


def _scat1(ref, pos, val, extra_mask=None):
    m = _iota() == 0
    if extra_mask is not None:
        m = m & extra_mask
    plsc.store_scatter(ref, [_bfull(pos, jnp.int32)], _bfull(val), mask=m)


def _nms_body(cls_hbm, bx_hbm, ksc_hbm, kid_hbm, kbx_hbm, kcnt_hbm,
              sc_v, x1_v, y1_v, x2_v, y2_v, l1_v, l2_v,
              kx1_v, ky1_v, kx2_v, ky2_v, kar_v, ksl_v, kil_v, cnt_v, sem):
    wid = lax.axis_index("s") * 2 + lax.axis_index("c")
    it16 = _iota()

    def do_task(t):
        b = t // _C
        cps = [
            pltpu.async_copy(
                cls_hbm.at[pl.ds(_al8(t * _N), _N)],
                sc_v.at[pl.ds(0, _N)], sem),
            pltpu.async_copy(
                bx_hbm.at[pl.ds(_al8(b * 4 * _N), _N)], x1_v, sem),
            pltpu.async_copy(
                bx_hbm.at[pl.ds(_al8((b * 4 + 1) * _N), _N)], y1_v, sem),
            pltpu.async_copy(
                bx_hbm.at[pl.ds(_al8((b * 4 + 2) * _N), _N)], x2_v, sem),
            pltpu.async_copy(
                bx_hbm.at[pl.ds(_al8((b * 4 + 3) * _N), _N)], y2_v, sem),
        ]
        for k in range((_SCPAD - _N) // 16):
            sc_v[pl.ds(_N + k * 16, 16)] = _bfull(_NEG)
        l1_v[pl.ds(_L1P, 16)] = _bfull(_NEG)

        def initk(k, _):
            for ref in (kx1_v, ky1_v, kx2_v, ky2_v, kar_v):
                ref[pl.ds(k * 16, 16)] = _bfull(jnp.float32(_BIG))
            return 0
        lax.fori_loop(0, _KCAP // 16, initk, 0)

        for cp in cps:
            cp.wait()

        def build(cidx):
            base = (cidx * 16 + it16) * 16
            mx = _bfull(_NEG)
            for j in range(16):
                mx = jnp.maximum(mx, plsc.load_gather(sc_v, [base + j]))
            mx = jnp.where(mx > jnp.float32(_TH), mx, _NEG)
            l1_v[pl.ds(cidx * 16, 16)] = mx
        plsc.parallel_loop(0, _NSG, 1, unroll=2)(build)

        def build2(c):
            base = (c * 16 + it16) * 16
            mx = _bfull(_NEG)
            for j in range(16):
                mx = jnp.maximum(mx, plsc.load_gather(l1_v, [base + j]))
            l2_v[pl.ds(c * 16, 16)] = mx
        plsc.parallel_loop(0, _L2P // 16, 1)(build2)

        def ex_cond(carry):
            return carry[2]

        def ex_body(carry):
            kc, it, _ = carry
            l2r = [l2_v[pl.ds(k * 16, 16)] for k in range(_L2P // 16)]
            mv = l2r[0]
            for k in range(1, _L2P // 16):
                mv = jnp.maximum(mv, l2r[k])
            m_sc = jnp.max(mv)

            def live(kc):
                g2 = jnp.int32(9 * _L2P)
                for k in range(_L2P // 16):
                    nk = _minwhere(l2r[k] == m_sc)
                    g2 = jnp.minimum(
                        g2, jnp.where(nk < 16, k * 16 + nk, 9 * _L2P))
                l1g = l1_v[pl.ds(g2 * 16, 16)]
                n1 = _minwhere(l1g == m_sc)
                g = g2 * 16 + n1
                sg = sc_v[pl.ds(g * 16, 16)]
                lane = _minwhere(sg == m_sc)
                f = g * 16 + lane
                idxv = _bfull(f, jnp.int32)
                cx1 = plsc.load_gather(x1_v, [idxv])
                cy1 = plsc.load_gather(y1_v, [idxv])
                cx2 = plsc.load_gather(x2_v, [idxv])
                cy2 = plsc.load_gather(y2_v, [idxv])
                ca = (cx2 - cx1) * (cy2 - cy1)

                nblk = (kc + 15) // 16

                def iou_body(j, acc):
                    a1 = kx1_v[pl.ds(j * 16, 16)]
                    b1 = ky1_v[pl.ds(j * 16, 16)]
                    a2 = kx2_v[pl.ds(j * 16, 16)]
                    b2 = ky2_v[pl.ds(j * 16, 16)]
                    ar = kar_v[pl.ds(j * 16, 16)]
                    iw = jnp.minimum(cx2, a2) - jnp.maximum(cx1, a1)
                    ih = jnp.minimum(cy2, b2) - jnp.maximum(cy1, b1)
                    inter = jnp.maximum(iw, 0.0) * jnp.maximum(ih, 0.0)
                    un = jnp.maximum(ar + ca - inter, jnp.float32(1e-8))
                    return acc | (inter > jnp.float32(0.5) * un)

                acc = plsc.parallel_loop(
                    0, nblk, 1, unroll=2, carry=it16 < 0)(iou_body)
                sup = plsc.all_reduce_population_count(acc)[0] > 0

                def keep(kc):
                    pos = _bfull(kc, jnp.int32)
                    m0 = it16 == 0
                    plsc.store_scatter(kx1_v, [pos], cx1, mask=m0)
                    plsc.store_scatter(ky1_v, [pos], cy1, mask=m0)
                    plsc.store_scatter(kx2_v, [pos], cx2, mask=m0)
                    plsc.store_scatter(ky2_v, [pos], cy2, mask=m0)
                    plsc.store_scatter(kar_v, [pos], ca, mask=m0)
                    plsc.store_scatter(ksl_v, [pos], _bfull(m_sc), mask=m0)
                    plsc.store_scatter(kil_v, [pos], idxv, mask=m0)
                    return kc + 1

                kc2 = lax.cond(sup, lambda k: k, keep, kc)

                sg2 = jnp.where(it16 == lane, _NEG, sg)
                sc_v[pl.ds(g * 16, 16)] = sg2
                gmr = jnp.max(sg2)
                gm = jnp.where(gmr > jnp.float32(_TH), gmr, _NEG)
                l1g2 = jnp.where(it16 == n1, gm, l1g)
                l1_v[pl.ds(g2 * 16, 16)] = l1g2
                q = (g2 // 16) * 16
                l2row = l2_v[pl.ds(q, 16)]
                l2row2 = jnp.where(it16 == (g2 % 16), jnp.max(l1g2), l2row)
                l2_v[pl.ds(q, 16)] = l2row2
                return kc2

            kc2 = lax.cond(m_sc > _NEG, live, lambda k: k, kc)
            go = (m_sc > _NEG) & (kc2 < _MAXD) & (it + 1 < _N)
            return kc2, it + 1, go

        kcf, _, _ = lax.while_loop(
            ex_cond, ex_body, (jnp.int32(0), jnp.int32(0), True))

        cnt_v[pl.ds(0, 16)] = _bfull(kcf, jnp.int32)
        pltpu.sync_copy(cnt_v, kcnt_hbm.at[pl.ds(_al8(t * 16), 16)])
        pltpu.sync_copy(ksl_v, ksc_hbm.at[pl.ds(_al8(t * _KCAP), _KCAP)])
        pltpu.sync_copy(kil_v, kid_hbm.at[pl.ds(_al8(t * _KCAP), _KCAP)])
        for j, ref in enumerate((kx1_v, ky1_v, kx2_v, ky2_v)):
            pltpu.sync_copy(
                ref, kbx_hbm.at[pl.ds(_al8((t * 4 + j) * _KCAP), _KCAP)])

    def tloop(i, _):
        t = wid + _NW * i

        @pl.when(t < _NT)
        def _():
            do_task(t)

        return 0

    lax.fori_loop(0, (_NT + _NW - 1) // _NW, tloop, 0)


_phase1 = pl.kernel(
    _nms_body,
    out_type=(
        jax.ShapeDtypeStruct((_NT * _KCAP,), jnp.float32),
        jax.ShapeDtypeStruct((_NT * _KCAP,), jnp.int32),
        jax.ShapeDtypeStruct((_NT * 4 * _KCAP,), jnp.float32),
        jax.ShapeDtypeStruct((_NT * 16,), jnp.int32),
    ),
    mesh=_mesh,
    compiler_params=_params,
    scratch_types=[
        pltpu.VMEM((_SCPAD,), jnp.float32),
        pltpu.VMEM((_N,), jnp.float32),
        pltpu.VMEM((_N,), jnp.float32),
        pltpu.VMEM((_N,), jnp.float32),
        pltpu.VMEM((_N,), jnp.float32),
        pltpu.VMEM((_L1PAD,), jnp.float32),
        pltpu.VMEM((_L2P,), jnp.float32),
        pltpu.VMEM((_KCAP,), jnp.float32),
        pltpu.VMEM((_KCAP,), jnp.float32),
        pltpu.VMEM((_KCAP,), jnp.float32),
        pltpu.VMEM((_KCAP,), jnp.float32),
        pltpu.VMEM((_KCAP,), jnp.float32),
        pltpu.VMEM((_KCAP,), jnp.float32),
        pltpu.VMEM((_KCAP,), jnp.int32),
        pltpu.VMEM((16,), jnp.int32),
        pltpu.SemaphoreType.DMA,
    ],
)


def _merge_body(ksc_hbm, kid_hbm, kbx_hbm, kcnt_hbm, lc_hbm,
                obox_hbm, osc_hbm, olab_hbm, olsc_hbm, ollab_hbm,
                ksc_v, kid_v, kbx_v, kcnt_v,
                obf_v, osc_v, olab_v, olsc_v, ollab_v,
                idxa_v, idxb_v, idxc_v, sub_v, lcr_v, sem):
    wid = lax.axis_index("s") * 2 + lax.axis_index("c")
    it16 = _iota()

    @pl.when(wid < _B)
    def _():
        b = wid
        cps = [
            pltpu.async_copy(
                ksc_hbm.at[pl.ds(_al8(b * _C * _KCAP), _C * _KCAP)],
                ksc_v, sem),
            pltpu.async_copy(
                kid_hbm.at[pl.ds(_al8(b * _C * _KCAP), _C * _KCAP)],
                kid_v, sem),
            pltpu.async_copy(
                kbx_hbm.at[pl.ds(_al8(b * _C * 4 * _KCAP), _C * 4 * _KCAP)],
                kbx_v, sem),
            pltpu.async_copy(
                kcnt_hbm.at[pl.ds(_al8(b * _C * 16), _C * 16)], kcnt_v, sem),
        ]
        for cp in cps:
            cp.wait()

        def initf(k, _):
            osc_v[pl.ds(k * 16, 16)] = _bfull(jnp.float32(-1.0))
            olsc_v[pl.ds(k * 16, 16)] = _bfull(jnp.float32(-1.0))
            olab_v[pl.ds(k * 16, 16)] = _bfull(jnp.int32(-1))
            ollab_v[pl.ds(k * 16, 16)] = _bfull(jnp.int32(-1))
            sub_v[pl.ds(k * 16, 16)] = _bfull(jnp.int32(0))
            return 0
        lax.fori_loop(0, _KCAP // 16, initf, 0)

        def initb(k, _):
            obf_v[pl.ds(k * 16, 16)] = _bfull(jnp.float32(-1.0))
            return 0
        lax.fori_loop(0, (_KCAP * 4) // 16, initb, 0)

        def initi(k, _):
            @pl.when(k < 8)
            def _():
                idxa_v[pl.ds(k * 16, 16)] = _bfull(jnp.int32(0))
                idxb_v[pl.ds(k * 16, 16)] = _bfull(jnp.int32(0))

            @pl.when(k < 3)
            def _():
                idxc_v[pl.ds(k * 16, 16)] = _bfull(jnp.int32(0))

            return 0
        lax.fori_loop(0, 8, initi, 0)

        zeros = _bfull(jnp.int32(0))
        row1 = jnp.minimum(it16 + 16, jnp.int32(_C - 1))
        cnt0 = plsc.load_gather(kcnt_v, [it16 * 16])
        cnt1 = plsc.load_gather(kcnt_v, [row1 * 16])
        lane_ok1 = it16 < (_C - 16)

        def mg_cond(carry):
            return carry[3]

        def mg_body(carry):
            r, h0, h1, _ = carry
            s0 = plsc.load_gather(
                ksc_v, [it16 * _KCAP + jnp.minimum(h0, _KCAP - 1)])
            s1 = plsc.load_gather(
                ksc_v, [row1 * _KCAP + jnp.minimum(h1, _KCAP - 1)])
            s0 = jnp.where(h0 < cnt0, s0, _NEG)
            s1 = jnp.where(lane_ok1 & (h1 < cnt1), s1, _NEG)
            m_sc = jnp.maximum(jnp.max(s0), jnp.max(s1))

            def live(op):
                r, h0, h1 = op
                c0 = _minwhere(s0 == m_sc)
                c1 = _minwhere(s1 == m_sc)
                cb = jnp.where(c0 < 16, c0, c1 + 16)
                lane = jnp.where(cb < 16, cb, cb - 16)
                hvec = jnp.where(cb < 16, h0, h1)
                hval = jnp.max(jnp.where(it16 == lane, hvec, 0))
                sel0 = (it16 == lane) & (cb < 16)
                sel1 = (it16 == lane) & (cb >= 16)
                h0n = jnp.where(sel0, h0 + 1, h0)
                h1n = jnp.where(sel1, h1 + 1, h1)

                _scat1(osc_v, r, m_sc)
                _scat1(olab_v, r, cb)
                crd = jnp.minimum(it16, jnp.int32(3))
                bxv = plsc.load_gather(
                    kbx_v, [(cb * 4 + crd) * _KCAP + hval])
                plsc.store_scatter(obf_v, [r * 4 + crd], bxv, mask=it16 < 4)
                fidx = plsc.load_gather(kid_v, [_bfull(cb * _KCAP + hval,
                                                       jnp.int32)])
                gidx = fidx + b * _N
                ridx = gidx // 8
                sub = gidx % 8
                m0 = it16 == 0
                _scat1(sub_v, r, sub)
                pa = _bfull(jnp.minimum(r, 127), jnp.int32)
                pb = _bfull(jnp.clip(r - 128, 0, 127), jnp.int32)
                pc = _bfull(jnp.clip(r - 256, 0, _KCAP - 257), jnp.int32)
                plsc.store_scatter(idxa_v, [pa], ridx, mask=m0 & (r < 128))
                plsc.store_scatter(idxb_v, [pb], ridx,
                                   mask=m0 & (r >= 128) & (r < 256))
                plsc.store_scatter(idxc_v, [pc], ridx, mask=m0 & (r >= 256))
                return r + 1, h0n, h1n

            go = m_sc > _NEG
            r2, h0n, h1n = lax.cond(go, live, lambda op: op, (r, h0, h1))
            return r2, h0n, h1n, go & (r2 < _MAXD)

        rf, _, _, _ = lax.while_loop(
            mg_cond, mg_body, (jnp.int32(0), zeros, zeros, True))

        g1 = pltpu.async_copy(lc_hbm.at[idxa_v], lcr_v.at[pl.ds(0, 128)], sem)
        g2 = pltpu.async_copy(
            lc_hbm.at[idxb_v], lcr_v.at[pl.ds(128, 128)], sem)
        g3 = pltpu.async_copy(
            lc_hbm.at[idxc_v], lcr_v.at[pl.ds(256, _KCAP - 256)], sem)
        g1.wait()
        g2.wait()
        g3.wait()

        def lcl(r, _):
            rv = _bfull(r, jnp.int32)
            sub16 = plsc.load_gather(sub_v, [rv])
            row = plsc.load_gather(lcr_v, [rv, sub16 * 16 + it16])
            lm = jnp.max(row)
            _scat1(olsc_v, r, lm)
            _scat1(ollab_v, r, _minwhere(row == lm))
            return 0
        lax.fori_loop(0, rf, lcl, 0)

        pltpu.sync_copy(
            obf_v, obox_hbm.at[pl.ds(_al8(b * _KCAP * 4), _KCAP * 4)])
        pltpu.sync_copy(osc_v, osc_hbm.at[pl.ds(_al8(b * _KCAP), _KCAP)])
        pltpu.sync_copy(olab_v, olab_hbm.at[pl.ds(_al8(b * _KCAP), _KCAP)])
        pltpu.sync_copy(olsc_v, olsc_hbm.at[pl.ds(_al8(b * _KCAP), _KCAP)])
        pltpu.sync_copy(ollab_v, ollab_hbm.at[pl.ds(_al8(b * _KCAP), _KCAP)])


_phase2 = pl.kernel(
    _merge_body,
    out_type=(
        jax.ShapeDtypeStruct((_B * _KCAP * 4,), jnp.float32),
        jax.ShapeDtypeStruct((_B * _KCAP,), jnp.float32),
        jax.ShapeDtypeStruct((_B * _KCAP,), jnp.int32),
        jax.ShapeDtypeStruct((_B * _KCAP,), jnp.float32),
        jax.ShapeDtypeStruct((_B * _KCAP,), jnp.int32),
    ),
    mesh=_mesh,
    compiler_params=_params,
    scratch_types=[
        pltpu.VMEM((_C * _KCAP,), jnp.float32),
        pltpu.VMEM((_C * _KCAP,), jnp.int32),
        pltpu.VMEM((_C * 4 * _KCAP,), jnp.float32),
        pltpu.VMEM((_C * 16,), jnp.int32),
        pltpu.VMEM((_KCAP * 4,), jnp.float32),
        pltpu.VMEM((_KCAP,), jnp.float32),
        pltpu.VMEM((_KCAP,), jnp.int32),
        pltpu.VMEM((_KCAP,), jnp.float32),
        pltpu.VMEM((_KCAP,), jnp.int32),
        pltpu.VMEM((128,), jnp.int32),
        pltpu.VMEM((128,), jnp.int32),
        pltpu.VMEM((_KCAP - 256,), jnp.int32),
        pltpu.VMEM((_KCAP,), jnp.int32),
        pltpu.VMEM((_KCAP, _LCROW), jnp.float32),
        pltpu.SemaphoreType.DMA,
    ],
)


@jax.jit
def kernel(boxes, classification, l_classification):
    cls_t = jnp.transpose(classification, (0, 2, 1)).reshape(-1)
    bx_t = jnp.transpose(boxes, (0, 2, 1)).reshape(-1)
    lc = jnp.pad(
        l_classification, ((0, 0), (0, 0), (0, 16 - _LC)),
        constant_values=-np.inf,
    ).reshape(_B * _N * 16 // _LCROW, _LCROW)
    ksc, kid, kbx, kcnt = _phase1(cls_t, bx_t)
    obox, osc, olab, olsc, ollab = _phase2(ksc, kid, kbx, kcnt, lc)
    return (
        obox.reshape(_B, _KCAP, 4)[:, :_MAXD],
        osc.reshape(_B, _KCAP)[:, :_MAXD],
        olab.reshape(_B, _KCAP)[:, :_MAXD],
        olsc.reshape(_B, _KCAP)[:, :_MAXD],
        ollab.reshape(_B, _KCAP)[:, :_MAXD],
    )


# speedup vs baseline: 1.0826x; 1.0826x over previous
import jax
import jax.numpy as jnp
import numpy as np
from jax import lax
from jax.experimental import pallas as pl
from jax.experimental.pallas import tpu as pltpu
from jax.experimental.pallas import tpu_sc as plsc

_B, _N, _C, _LC = 4, 20000, 20, 10
_TH = 0.05
_MAXD = 300
_KCAP = 304
_NG = _N // 16
_NSG = (_NG + 15) // 16
_L1P = _NSG * 16
_L2P = 80
_NT = _B * _C
_NW = 32
_NEG = float("-inf")
_LCROW = 128

_mesh = plsc.VectorSubcoreMesh(
    core_axis_name="c", subcore_axis_name="s", num_cores=2, num_subcores=16
)
_params = pltpu.CompilerParams(needs_layout_passes=False)


_SCPAD = _L1P * 16
_L1PAD = _L2P * 16
_BIG = 3.0e38


def _iota():
    return lax.iota(jnp.int32, 16)


def _minwhere(mask):
    return plsc.all_reduce_ffs(mask)[0]


def _bfull(x, dtype=None):
    v = jnp.full((16,), x)
    return v if dtype is None else v.astype(dtype)


def _al8(x):
    return pl.multiple_of(x, 8)


def _vmaxsplat(v):
    i = _iota()
    for sh in (8, 4, 2, 1):
        v = jnp.maximum(v, jnp.take(v, i ^ sh))
    return v


def _scat1(ref, pos, val, extra_mask=None):
    m = _iota() == 0
    if extra_mask is not None:
        m = m & extra_mask
    plsc.store_scatter(ref, [_bfull(pos, jnp.int32)], _bfull(val), mask=m)


def _nms_body(cls_hbm, bx_hbm, ksc_hbm, kid_hbm, kbx_hbm, kcnt_hbm,
              sc_v, x1_v, y1_v, x2_v, y2_v, l1_v, l2_v,
              kx1_v, ky1_v, kx2_v, ky2_v, kar_v, ksl_v, kil_v, cnt_v, sem):
    wid = lax.axis_index("s") * 2 + lax.axis_index("c")
    it16 = _iota()

    def do_task(t):
        b = t // _C
        cps = [
            pltpu.async_copy(
                cls_hbm.at[pl.ds(_al8(t * _N), _N)],
                sc_v.at[pl.ds(0, _N)], sem),
            pltpu.async_copy(
                bx_hbm.at[pl.ds(_al8(b * 4 * _N), _N)], x1_v, sem),
            pltpu.async_copy(
                bx_hbm.at[pl.ds(_al8((b * 4 + 1) * _N), _N)], y1_v, sem),
            pltpu.async_copy(
                bx_hbm.at[pl.ds(_al8((b * 4 + 2) * _N), _N)], x2_v, sem),
            pltpu.async_copy(
                bx_hbm.at[pl.ds(_al8((b * 4 + 3) * _N), _N)], y2_v, sem),
        ]
        for k in range((_SCPAD - _N) // 16):
            sc_v[pl.ds(_N + k * 16, 16)] = _bfull(_NEG)
        l1_v[pl.ds(_L1P, 16)] = _bfull(_NEG)

        def initk(k, _):
            for ref in (kx1_v, ky1_v, kx2_v, ky2_v, kar_v):
                ref[pl.ds(k * 16, 16)] = _bfull(jnp.float32(_BIG))
            return 0
        lax.fori_loop(0, _KCAP // 16, initk, 0)

        for cp in cps:
            cp.wait()

        def build(cidx):
            base = (cidx * 16 + it16) * 16
            mx = _bfull(_NEG)
            for j in range(16):
                mx = jnp.maximum(mx, plsc.load_gather(sc_v, [base + j]))
            mx = jnp.where(mx > jnp.float32(_TH), mx, _NEG)
            l1_v[pl.ds(cidx * 16, 16)] = mx
        plsc.parallel_loop(0, _NSG, 1, unroll=2)(build)

        def build2(c):
            base = (c * 16 + it16) * 16
            mx = _bfull(_NEG)
            for j in range(16):
                mx = jnp.maximum(mx, plsc.load_gather(l1_v, [base + j]))
            l2_v[pl.ds(c * 16, 16)] = mx
        plsc.parallel_loop(0, _L2P // 16, 1)(build2)

        def ex_cond(carry):
            return carry[3]

        def ex_body(carry):
            kc, kcv, it, _ = carry
            l2r = [l2_v[pl.ds(k * 16, 16)] for k in range(_L2P // 16)]
            mv = l2r[0]
            for k in range(1, _L2P // 16):
                mv = jnp.maximum(mv, l2r[k])
            msp = _vmaxsplat(mv)
            m_ok = msp[0] > _NEG

            def live(op):
                kc, kcv = op
                g2v = _bfull(jnp.int32(9 * _L2P))
                for k in range(_L2P // 16):
                    nkv = plsc.all_reduce_ffs(l2r[k] == msp)
                    g2v = jnp.minimum(
                        g2v, jnp.where(nkv < 16, k * 16 + nkv, 9 * _L2P))
                l1g = plsc.load_gather(l1_v, [g2v * 16 + it16])
                n1v = plsc.all_reduce_ffs(l1g == msp)
                gv = g2v * 16 + n1v
                sg = plsc.load_gather(sc_v, [gv * 16 + it16])
                lanev = plsc.all_reduce_ffs(sg == msp)
                fv = gv * 16 + lanev
                cx1 = plsc.load_gather(x1_v, [fv])
                cy1 = plsc.load_gather(y1_v, [fv])
                cx2 = plsc.load_gather(x2_v, [fv])
                cy2 = plsc.load_gather(y2_v, [fv])
                ca = (cx2 - cx1) * (cy2 - cy1)

                nblk = (kc + 15) // 16

                def iou_body(j, acc):
                    a1 = kx1_v[pl.ds(j * 16, 16)]
                    b1 = ky1_v[pl.ds(j * 16, 16)]
                    a2 = kx2_v[pl.ds(j * 16, 16)]
                    b2 = ky2_v[pl.ds(j * 16, 16)]
                    ar = kar_v[pl.ds(j * 16, 16)]
                    iw = jnp.minimum(cx2, a2) - jnp.maximum(cx1, a1)
                    ih = jnp.minimum(cy2, b2) - jnp.maximum(cy1, b1)
                    inter = jnp.maximum(iw, 0.0) * jnp.maximum(ih, 0.0)
                    un = jnp.maximum(ar + ca - inter, jnp.float32(1e-8))
                    return acc | (inter > jnp.float32(0.5) * un)

                acc = plsc.parallel_loop(
                    0, nblk, 1, unroll=2, carry=it16 < 0)(iou_body)
                popc = plsc.all_reduce_population_count(acc)
                supv = popc > 0

                keepm = (it16 == 0) & (~supv)
                plsc.store_scatter(kx1_v, [kcv], cx1, mask=keepm)
                plsc.store_scatter(ky1_v, [kcv], cy1, mask=keepm)
                plsc.store_scatter(kx2_v, [kcv], cx2, mask=keepm)
                plsc.store_scatter(ky2_v, [kcv], cy2, mask=keepm)
                plsc.store_scatter(kar_v, [kcv], ca, mask=keepm)
                plsc.store_scatter(ksl_v, [kcv], msp, mask=keepm)
                plsc.store_scatter(kil_v, [kcv], fv, mask=keepm)
                kcv2 = jnp.where(supv, kcv, kcv + 1)
                kc2 = kc + jnp.where(popc[0] > 0, 0, 1)

                sg2 = jnp.where(it16 == lanev, _NEG, sg)
                plsc.store_scatter(sc_v, [gv * 16 + it16], sg2)
                gmr = _vmaxsplat(sg2)
                gm = jnp.where(gmr > jnp.float32(_TH), gmr, _NEG)
                l1g2 = jnp.where(it16 == n1v, gm, l1g)
                plsc.store_scatter(l1_v, [g2v * 16 + it16], l1g2)
                qv = (g2v & ~jnp.int32(15)) + it16
                l2row = plsc.load_gather(l2_v, [qv])
                l2row2 = jnp.where(
                    it16 == (g2v & jnp.int32(15)), _vmaxsplat(l1g2), l2row)
                plsc.store_scatter(l2_v, [qv], l2row2)
                return kc2, kcv2

            kc2, kcv2 = lax.cond(m_ok, live, lambda op: op, (kc, kcv))
            go = m_ok & (kc2 < _MAXD) & (it + 1 < _N)
            return kc2, kcv2, it + 1, go

        kcf, _, _, _ = lax.while_loop(
            ex_cond, ex_body,
            (jnp.int32(0), _bfull(jnp.int32(0)), jnp.int32(0), True))

        cnt_v[pl.ds(0, 16)] = _bfull(kcf, jnp.int32)
        pltpu.sync_copy(cnt_v, kcnt_hbm.at[pl.ds(_al8(t * 16), 16)])
        pltpu.sync_copy(ksl_v, ksc_hbm.at[pl.ds(_al8(t * _KCAP), _KCAP)])
        pltpu.sync_copy(kil_v, kid_hbm.at[pl.ds(_al8(t * _KCAP), _KCAP)])
        for j, ref in enumerate((kx1_v, ky1_v, kx2_v, ky2_v)):
            pltpu.sync_copy(
                ref, kbx_hbm.at[pl.ds(_al8((t * 4 + j) * _KCAP), _KCAP)])

    def tloop(i, _):
        t = wid + _NW * i

        @pl.when(t < _NT)
        def _():
            do_task(t)

        return 0

    lax.fori_loop(0, (_NT + _NW - 1) // _NW, tloop, 0)


_phase1 = pl.kernel(
    _nms_body,
    out_type=(
        jax.ShapeDtypeStruct((_NT * _KCAP,), jnp.float32),
        jax.ShapeDtypeStruct((_NT * _KCAP,), jnp.int32),
        jax.ShapeDtypeStruct((_NT * 4 * _KCAP,), jnp.float32),
        jax.ShapeDtypeStruct((_NT * 16,), jnp.int32),
    ),
    mesh=_mesh,
    compiler_params=_params,
    scratch_types=[
        pltpu.VMEM((_SCPAD,), jnp.float32),
        pltpu.VMEM((_N,), jnp.float32),
        pltpu.VMEM((_N,), jnp.float32),
        pltpu.VMEM((_N,), jnp.float32),
        pltpu.VMEM((_N,), jnp.float32),
        pltpu.VMEM((_L1PAD,), jnp.float32),
        pltpu.VMEM((_L2P,), jnp.float32),
        pltpu.VMEM((_KCAP,), jnp.float32),
        pltpu.VMEM((_KCAP,), jnp.float32),
        pltpu.VMEM((_KCAP,), jnp.float32),
        pltpu.VMEM((_KCAP,), jnp.float32),
        pltpu.VMEM((_KCAP,), jnp.float32),
        pltpu.VMEM((_KCAP,), jnp.float32),
        pltpu.VMEM((_KCAP,), jnp.int32),
        pltpu.VMEM((16,), jnp.int32),
        pltpu.SemaphoreType.DMA,
    ],
)


def _merge_body(ksc_hbm, kid_hbm, kbx_hbm, kcnt_hbm, lc_hbm,
                obox_hbm, osc_hbm, olab_hbm, olsc_hbm, ollab_hbm,
                ksc_v, kid_v, kbx_v, kcnt_v,
                obf_v, osc_v, olab_v, olsc_v, ollab_v,
                idxa_v, idxb_v, idxc_v, sub_v, lcr_v, sem):
    wid = lax.axis_index("s") * 2 + lax.axis_index("c")
    it16 = _iota()

    @pl.when(wid < _B)
    def _():
        b = wid
        cps = [
            pltpu.async_copy(
                ksc_hbm.at[pl.ds(_al8(b * _C * _KCAP), _C * _KCAP)],
                ksc_v, sem),
            pltpu.async_copy(
                kid_hbm.at[pl.ds(_al8(b * _C * _KCAP), _C * _KCAP)],
                kid_v, sem),
            pltpu.async_copy(
                kbx_hbm.at[pl.ds(_al8(b * _C * 4 * _KCAP), _C * 4 * _KCAP)],
                kbx_v, sem),
            pltpu.async_copy(
                kcnt_hbm.at[pl.ds(_al8(b * _C * 16), _C * 16)], kcnt_v, sem),
        ]
        for cp in cps:
            cp.wait()

        def initf(k, _):
            osc_v[pl.ds(k * 16, 16)] = _bfull(jnp.float32(-1.0))
            olsc_v[pl.ds(k * 16, 16)] = _bfull(jnp.float32(-1.0))
            olab_v[pl.ds(k * 16, 16)] = _bfull(jnp.int32(-1))
            ollab_v[pl.ds(k * 16, 16)] = _bfull(jnp.int32(-1))
            sub_v[pl.ds(k * 16, 16)] = _bfull(jnp.int32(0))
            return 0
        lax.fori_loop(0, _KCAP // 16, initf, 0)

        def initb(k, _):
            obf_v[pl.ds(k * 16, 16)] = _bfull(jnp.float32(-1.0))
            return 0
        lax.fori_loop(0, (_KCAP * 4) // 16, initb, 0)

        def initi(k, _):
            @pl.when(k < 8)
            def _():
                idxa_v[pl.ds(k * 16, 16)] = _bfull(jnp.int32(0))
                idxb_v[pl.ds(k * 16, 16)] = _bfull(jnp.int32(0))

            @pl.when(k < 3)
            def _():
                idxc_v[pl.ds(k * 16, 16)] = _bfull(jnp.int32(0))

            return 0
        lax.fori_loop(0, 8, initi, 0)

        zeros = _bfull(jnp.int32(0))
        row1 = jnp.minimum(it16 + 16, jnp.int32(_C - 1))
        cnt0 = plsc.load_gather(kcnt_v, [it16 * 16])
        cnt1 = plsc.load_gather(kcnt_v, [row1 * 16])
        lane_ok1 = it16 < (_C - 16)

        def mg_cond(carry):
            return carry[3]

        def mg_body(carry):
            r, h0, h1, _ = carry
            s0 = plsc.load_gather(
                ksc_v, [it16 * _KCAP + jnp.minimum(h0, _KCAP - 1)])
            s1 = plsc.load_gather(
                ksc_v, [row1 * _KCAP + jnp.minimum(h1, _KCAP - 1)])
            s0 = jnp.where(h0 < cnt0, s0, _NEG)
            s1 = jnp.where(lane_ok1 & (h1 < cnt1), s1, _NEG)
            m_sc = jnp.maximum(jnp.max(s0), jnp.max(s1))

            def live(op):
                r, h0, h1 = op
                c0 = _minwhere(s0 == m_sc)
                c1 = _minwhere(s1 == m_sc)
                cb = jnp.where(c0 < 16, c0, c1 + 16)
                lane = jnp.where(cb < 16, cb, cb - 16)
                hvec = jnp.where(cb < 16, h0, h1)
                hval = jnp.max(jnp.where(it16 == lane, hvec, 0))
                sel0 = (it16 == lane) & (cb < 16)
                sel1 = (it16 == lane) & (cb >= 16)
                h0n = jnp.where(sel0, h0 + 1, h0)
                h1n = jnp.where(sel1, h1 + 1, h1)

                _scat1(osc_v, r, m_sc)
                _scat1(olab_v, r, cb)
                crd = jnp.minimum(it16, jnp.int32(3))
                bxv = plsc.load_gather(
                    kbx_v, [(cb * 4 + crd) * _KCAP + hval])
                plsc.store_scatter(obf_v, [r * 4 + crd], bxv, mask=it16 < 4)
                fidx = plsc.load_gather(kid_v, [_bfull(cb * _KCAP + hval,
                                                       jnp.int32)])
                gidx = fidx + b * _N
                ridx = gidx // 8
                sub = gidx % 8
                m0 = it16 == 0
                _scat1(sub_v, r, sub)
                pa = _bfull(jnp.minimum(r, 127), jnp.int32)
                pb = _bfull(jnp.clip(r - 128, 0, 127), jnp.int32)
                pc = _bfull(jnp.clip(r - 256, 0, _KCAP - 257), jnp.int32)
                plsc.store_scatter(idxa_v, [pa], ridx, mask=m0 & (r < 128))
                plsc.store_scatter(idxb_v, [pb], ridx,
                                   mask=m0 & (r >= 128) & (r < 256))
                plsc.store_scatter(idxc_v, [pc], ridx, mask=m0 & (r >= 256))
                return r + 1, h0n, h1n

            go = m_sc > _NEG
            r2, h0n, h1n = lax.cond(go, live, lambda op: op, (r, h0, h1))
            return r2, h0n, h1n, go & (r2 < _MAXD)

        rf, _, _, _ = lax.while_loop(
            mg_cond, mg_body, (jnp.int32(0), zeros, zeros, True))

        g1 = pltpu.async_copy(lc_hbm.at[idxa_v], lcr_v.at[pl.ds(0, 128)], sem)
        g2 = pltpu.async_copy(
            lc_hbm.at[idxb_v], lcr_v.at[pl.ds(128, 128)], sem)
        g3 = pltpu.async_copy(
            lc_hbm.at[idxc_v], lcr_v.at[pl.ds(256, _KCAP - 256)], sem)
        g1.wait()
        g2.wait()
        g3.wait()

        def lcl(r, _):
            rv = _bfull(r, jnp.int32)
            sub16 = plsc.load_gather(sub_v, [rv])
            row = plsc.load_gather(lcr_v, [rv, sub16 * 16 + it16])
            lm = jnp.max(row)
            _scat1(olsc_v, r, lm)
            _scat1(ollab_v, r, _minwhere(row == lm))
            return 0
        lax.fori_loop(0, rf, lcl, 0)

        pltpu.sync_copy(
            obf_v, obox_hbm.at[pl.ds(_al8(b * _KCAP * 4), _KCAP * 4)])
        pltpu.sync_copy(osc_v, osc_hbm.at[pl.ds(_al8(b * _KCAP), _KCAP)])
        pltpu.sync_copy(olab_v, olab_hbm.at[pl.ds(_al8(b * _KCAP), _KCAP)])
        pltpu.sync_copy(olsc_v, olsc_hbm.at[pl.ds(_al8(b * _KCAP), _KCAP)])
        pltpu.sync_copy(ollab_v, ollab_hbm.at[pl.ds(_al8(b * _KCAP), _KCAP)])


_phase2 = pl.kernel(
    _merge_body,
    out_type=(
        jax.ShapeDtypeStruct((_B * _KCAP * 4,), jnp.float32),
        jax.ShapeDtypeStruct((_B * _KCAP,), jnp.float32),
        jax.ShapeDtypeStruct((_B * _KCAP,), jnp.int32),
        jax.ShapeDtypeStruct((_B * _KCAP,), jnp.float32),
        jax.ShapeDtypeStruct((_B * _KCAP,), jnp.int32),
    ),
    mesh=_mesh,
    compiler_params=_params,
    scratch_types=[
        pltpu.VMEM((_C * _KCAP,), jnp.float32),
        pltpu.VMEM((_C * _KCAP,), jnp.int32),
        pltpu.VMEM((_C * 4 * _KCAP,), jnp.float32),
        pltpu.VMEM((_C * 16,), jnp.int32),
        pltpu.VMEM((_KCAP * 4,), jnp.float32),
        pltpu.VMEM((_KCAP,), jnp.float32),
        pltpu.VMEM((_KCAP,), jnp.int32),
        pltpu.VMEM((_KCAP,), jnp.float32),
        pltpu.VMEM((_KCAP,), jnp.int32),
        pltpu.VMEM((128,), jnp.int32),
        pltpu.VMEM((128,), jnp.int32),
        pltpu.VMEM((_KCAP - 256,), jnp.int32),
        pltpu.VMEM((_KCAP,), jnp.int32),
        pltpu.VMEM((_KCAP, _LCROW), jnp.float32),
        pltpu.SemaphoreType.DMA,
    ],
)


@jax.jit
def kernel(boxes, classification, l_classification):
    cls_t = jnp.transpose(classification, (0, 2, 1)).reshape(-1)
    bx_t = jnp.transpose(boxes, (0, 2, 1)).reshape(-1)
    lc = jnp.pad(
        l_classification, ((0, 0), (0, 0), (0, 16 - _LC)),
        constant_values=-np.inf,
    ).reshape(_B * _N * 16 // _LCROW, _LCROW)
    ksc, kid, kbx, kcnt = _phase1(cls_t, bx_t)
    obox, osc, olab, olsc, ollab = _phase2(ksc, kid, kbx, kcnt, lc)
    return (
        obox.reshape(_B, _KCAP, 4)[:, :_MAXD],
        osc.reshape(_B, _KCAP)[:, :_MAXD],
        olab.reshape(_B, _KCAP)[:, :_MAXD],
        olsc.reshape(_B, _KCAP)[:, :_MAXD],
        ollab.reshape(_B, _KCAP)[:, :_MAXD],
    )


# speedup vs baseline: 1.0935x; 1.0101x over previous
import jax
import jax.numpy as jnp
import numpy as np
from jax import lax
from jax.experimental import pallas as pl
from jax.experimental.pallas import tpu as pltpu
from jax.experimental.pallas import tpu_sc as plsc

_B, _N, _C, _LC = 4, 20000, 20, 10
_TH = 0.05
_MAXD = 300
_KCAP = 304
_NG = _N // 16
_NSG = (_NG + 15) // 16
_L1P = _NSG * 16
_L2P = 80
_NT = _B * _C
_NW = 32
_NEG = float("-inf")
_LCROW = 128

_mesh = plsc.VectorSubcoreMesh(
    core_axis_name="c", subcore_axis_name="s", num_cores=2, num_subcores=16
)
_params = pltpu.CompilerParams(needs_layout_passes=False)


_SCPAD = _L1P * 16
_L1PAD = _L2P * 16
_BIG = 3.0e38


def _iota():
    return lax.iota(jnp.int32, 16)


def _minwhere(mask):
    return plsc.all_reduce_ffs(mask)[0]


def _bfull(x, dtype=None):
    v = jnp.full((16,), x)
    return v if dtype is None else v.astype(dtype)


def _al8(x):
    return pl.multiple_of(x, 8)


def _vmaxsplat(v):
    i = _iota()
    for sh in (8, 4, 2, 1):
        v = jnp.maximum(v, jnp.take(v, i ^ sh))
    return v


def _scat1(ref, pos, val, extra_mask=None):
    m = _iota() == 0
    if extra_mask is not None:
        m = m & extra_mask
    plsc.store_scatter(ref, [_bfull(pos, jnp.int32)], _bfull(val), mask=m)


def _nms_body(cls_hbm, bx_hbm, ksc_hbm, kid_hbm, kbx_hbm, kcnt_hbm,
              sc_v, x1_v, y1_v, x2_v, y2_v, l1_v,
              kx1_v, ky1_v, kx2_v, ky2_v, kar_v, ksl_v, kil_v, cnt_v, sem):
    wid = lax.axis_index("s") * 2 + lax.axis_index("c")
    it16 = _iota()

    def do_task(t):
        b = t // _C
        cps = [
            pltpu.async_copy(
                cls_hbm.at[pl.ds(_al8(t * _N), _N)],
                sc_v.at[pl.ds(0, _N)], sem),
            pltpu.async_copy(
                bx_hbm.at[pl.ds(_al8(b * 4 * _N), _N)], x1_v, sem),
            pltpu.async_copy(
                bx_hbm.at[pl.ds(_al8((b * 4 + 1) * _N), _N)], y1_v, sem),
            pltpu.async_copy(
                bx_hbm.at[pl.ds(_al8((b * 4 + 2) * _N), _N)], x2_v, sem),
            pltpu.async_copy(
                bx_hbm.at[pl.ds(_al8((b * 4 + 3) * _N), _N)], y2_v, sem),
        ]
        for k in range((_SCPAD - _N) // 16):
            sc_v[pl.ds(_N + k * 16, 16)] = _bfull(_NEG)
        l1_v[pl.ds(_L1P, 16)] = _bfull(_NEG)

        def initk(k, _):
            for ref in (kx1_v, ky1_v, kx2_v, ky2_v, kar_v):
                ref[pl.ds(k * 16, 16)] = _bfull(jnp.float32(_BIG))
            return 0
        lax.fori_loop(0, _KCAP // 16, initk, 0)

        for cp in cps:
            cp.wait()

        def build(cidx):
            base = (cidx * 16 + it16) * 16
            mx = _bfull(_NEG)
            for j in range(16):
                mx = jnp.maximum(mx, plsc.load_gather(sc_v, [base + j]))
            mx = jnp.where(mx > jnp.float32(_TH), mx, _NEG)
            l1_v[pl.ds(cidx * 16, 16)] = mx
        plsc.parallel_loop(0, _NSG, 1, unroll=2)(build)

        l2init = []
        for c in range(_L2P // 16):
            base = (c * 16 + it16) * 16
            mx = _bfull(_NEG)
            for j in range(16):
                mx = jnp.maximum(mx, plsc.load_gather(l1_v, [base + j]))
            l2init.append(mx)
        l2init = tuple(l2init)

        def ex_cond(carry):
            return carry[3]

        def ex_body(carry):
            kc, kcv, it, _, l2r = carry
            mv = l2r[0]
            for k in range(1, _L2P // 16):
                mv = jnp.maximum(mv, l2r[k])
            msp = _vmaxsplat(mv)
            m_ok = msp[0] > _NEG

            def live(op):
                kc, kcv, l2r = op
                g2v = _bfull(jnp.int32(9 * _L2P))
                for k in range(_L2P // 16):
                    nkv = plsc.all_reduce_ffs(l2r[k] == msp)
                    g2v = jnp.minimum(
                        g2v, jnp.where(nkv < 16, k * 16 + nkv, 9 * _L2P))
                l1g = plsc.load_gather(l1_v, [g2v * 16 + it16])
                n1v = plsc.all_reduce_ffs(l1g == msp)
                gv = g2v * 16 + n1v
                sg = plsc.load_gather(sc_v, [gv * 16 + it16])
                lanev = plsc.all_reduce_ffs(sg == msp)
                fv = gv * 16 + lanev
                cx1 = plsc.load_gather(x1_v, [fv])
                cy1 = plsc.load_gather(y1_v, [fv])
                cx2 = plsc.load_gather(x2_v, [fv])
                cy2 = plsc.load_gather(y2_v, [fv])
                ca = (cx2 - cx1) * (cy2 - cy1)

                nblk = (kc + 15) // 16

                def iou_body(j, acc):
                    a1 = kx1_v[pl.ds(j * 16, 16)]
                    b1 = ky1_v[pl.ds(j * 16, 16)]
                    a2 = kx2_v[pl.ds(j * 16, 16)]
                    b2 = ky2_v[pl.ds(j * 16, 16)]
                    ar = kar_v[pl.ds(j * 16, 16)]
                    iw = jnp.minimum(cx2, a2) - jnp.maximum(cx1, a1)
                    ih = jnp.minimum(cy2, b2) - jnp.maximum(cy1, b1)
                    inter = jnp.maximum(iw, 0.0) * jnp.maximum(ih, 0.0)
                    un = jnp.maximum(ar + ca - inter, jnp.float32(1e-8))
                    return acc | (inter > jnp.float32(0.5) * un)

                acc = plsc.parallel_loop(
                    0, nblk, 1, unroll=2, carry=it16 < 0)(iou_body)
                popc = plsc.all_reduce_population_count(acc)
                supv = popc > 0

                keepm = (it16 == 0) & (~supv)
                plsc.store_scatter(kx1_v, [kcv], cx1, mask=keepm)
                plsc.store_scatter(ky1_v, [kcv], cy1, mask=keepm)
                plsc.store_scatter(kx2_v, [kcv], cx2, mask=keepm)
                plsc.store_scatter(ky2_v, [kcv], cy2, mask=keepm)
                plsc.store_scatter(kar_v, [kcv], ca, mask=keepm)
                plsc.store_scatter(ksl_v, [kcv], msp, mask=keepm)
                plsc.store_scatter(kil_v, [kcv], fv, mask=keepm)
                kcv2 = jnp.where(supv, kcv, kcv + 1)
                kc2 = kc + jnp.where(popc[0] > 0, 0, 1)

                sg2 = jnp.where(it16 == lanev, _NEG, sg)
                plsc.store_scatter(sc_v, [gv * 16 + it16], sg2)
                gmr = _vmaxsplat(sg2)
                gm = jnp.where(gmr > jnp.float32(_TH), gmr, _NEG)
                l1g2 = jnp.where(it16 == n1v, gm, l1g)
                plsc.store_scatter(l1_v, [g2v * 16 + it16], l1g2)
                sgm = _vmaxsplat(l1g2)
                kvreg = g2v >> 4
                klane = g2v & jnp.int32(15)
                l2n = tuple(
                    jnp.where((kvreg == k) & (it16 == klane), sgm, l2r[k])
                    for k in range(_L2P // 16))
                return kc2, kcv2, l2n

            kc2, kcv2, l2n = lax.cond(
                m_ok, live, lambda op: op, (kc, kcv, l2r))
            go = m_ok & (kc2 < _MAXD) & (it + 1 < _N)
            return kc2, kcv2, it + 1, go, l2n

        kcf, _, _, _, _ = lax.while_loop(
            ex_cond, ex_body,
            (jnp.int32(0), _bfull(jnp.int32(0)), jnp.int32(0), True, l2init))

        cnt_v[pl.ds(0, 16)] = _bfull(kcf, jnp.int32)
        pltpu.sync_copy(cnt_v, kcnt_hbm.at[pl.ds(_al8(t * 16), 16)])
        pltpu.sync_copy(ksl_v, ksc_hbm.at[pl.ds(_al8(t * _KCAP), _KCAP)])
        pltpu.sync_copy(kil_v, kid_hbm.at[pl.ds(_al8(t * _KCAP), _KCAP)])
        for j, ref in enumerate((kx1_v, ky1_v, kx2_v, ky2_v)):
            pltpu.sync_copy(
                ref, kbx_hbm.at[pl.ds(_al8((t * 4 + j) * _KCAP), _KCAP)])

    def tloop(i, _):
        t = wid + _NW * i

        @pl.when(t < _NT)
        def _():
            do_task(t)

        return 0

    lax.fori_loop(0, (_NT + _NW - 1) // _NW, tloop, 0)


_phase1 = pl.kernel(
    _nms_body,
    out_type=(
        jax.ShapeDtypeStruct((_NT * _KCAP,), jnp.float32),
        jax.ShapeDtypeStruct((_NT * _KCAP,), jnp.int32),
        jax.ShapeDtypeStruct((_NT * 4 * _KCAP,), jnp.float32),
        jax.ShapeDtypeStruct((_NT * 16,), jnp.int32),
    ),
    mesh=_mesh,
    compiler_params=_params,
    scratch_types=[
        pltpu.VMEM((_SCPAD,), jnp.float32),
        pltpu.VMEM((_N,), jnp.float32),
        pltpu.VMEM((_N,), jnp.float32),
        pltpu.VMEM((_N,), jnp.float32),
        pltpu.VMEM((_N,), jnp.float32),
        pltpu.VMEM((_L1PAD,), jnp.float32),
        pltpu.VMEM((_KCAP,), jnp.float32),
        pltpu.VMEM((_KCAP,), jnp.float32),
        pltpu.VMEM((_KCAP,), jnp.float32),
        pltpu.VMEM((_KCAP,), jnp.float32),
        pltpu.VMEM((_KCAP,), jnp.float32),
        pltpu.VMEM((_KCAP,), jnp.float32),
        pltpu.VMEM((_KCAP,), jnp.int32),
        pltpu.VMEM((16,), jnp.int32),
        pltpu.SemaphoreType.DMA,
    ],
)


def _merge_body(ksc_hbm, kid_hbm, kbx_hbm, kcnt_hbm, lc_hbm,
                obox_hbm, osc_hbm, olab_hbm, olsc_hbm, ollab_hbm,
                ksc_v, kid_v, kbx_v, kcnt_v,
                obf_v, osc_v, olab_v, olsc_v, ollab_v,
                idxa_v, idxb_v, idxc_v, sub_v, lcr_v, sem):
    wid = lax.axis_index("s") * 2 + lax.axis_index("c")
    it16 = _iota()

    @pl.when(wid < _B)
    def _():
        b = wid
        cps = [
            pltpu.async_copy(
                ksc_hbm.at[pl.ds(_al8(b * _C * _KCAP), _C * _KCAP)],
                ksc_v, sem),
            pltpu.async_copy(
                kid_hbm.at[pl.ds(_al8(b * _C * _KCAP), _C * _KCAP)],
                kid_v, sem),
            pltpu.async_copy(
                kbx_hbm.at[pl.ds(_al8(b * _C * 4 * _KCAP), _C * 4 * _KCAP)],
                kbx_v, sem),
            pltpu.async_copy(
                kcnt_hbm.at[pl.ds(_al8(b * _C * 16), _C * 16)], kcnt_v, sem),
        ]
        for cp in cps:
            cp.wait()

        def initf(k, _):
            osc_v[pl.ds(k * 16, 16)] = _bfull(jnp.float32(-1.0))
            olsc_v[pl.ds(k * 16, 16)] = _bfull(jnp.float32(-1.0))
            olab_v[pl.ds(k * 16, 16)] = _bfull(jnp.int32(-1))
            ollab_v[pl.ds(k * 16, 16)] = _bfull(jnp.int32(-1))
            sub_v[pl.ds(k * 16, 16)] = _bfull(jnp.int32(0))
            return 0
        lax.fori_loop(0, _KCAP // 16, initf, 0)

        def initb(k, _):
            obf_v[pl.ds(k * 16, 16)] = _bfull(jnp.float32(-1.0))
            return 0
        lax.fori_loop(0, (_KCAP * 4) // 16, initb, 0)

        def initi(k, _):
            @pl.when(k < 8)
            def _():
                idxa_v[pl.ds(k * 16, 16)] = _bfull(jnp.int32(0))
                idxb_v[pl.ds(k * 16, 16)] = _bfull(jnp.int32(0))

            @pl.when(k < 3)
            def _():
                idxc_v[pl.ds(k * 16, 16)] = _bfull(jnp.int32(0))

            return 0
        lax.fori_loop(0, 8, initi, 0)

        zeros = _bfull(jnp.int32(0))
        row1 = jnp.minimum(it16 + 16, jnp.int32(_C - 1))
        cnt0 = plsc.load_gather(kcnt_v, [it16 * 16])
        cnt1 = plsc.load_gather(kcnt_v, [row1 * 16])
        lane_ok1 = it16 < (_C - 16)

        def mg_cond(carry):
            return carry[3]

        def mg_body(carry):
            r, h0, h1, _ = carry
            s0 = plsc.load_gather(
                ksc_v, [it16 * _KCAP + jnp.minimum(h0, _KCAP - 1)])
            s1 = plsc.load_gather(
                ksc_v, [row1 * _KCAP + jnp.minimum(h1, _KCAP - 1)])
            s0 = jnp.where(h0 < cnt0, s0, _NEG)
            s1 = jnp.where(lane_ok1 & (h1 < cnt1), s1, _NEG)
            m_sc = jnp.maximum(jnp.max(s0), jnp.max(s1))

            def live(op):
                r, h0, h1 = op
                c0 = _minwhere(s0 == m_sc)
                c1 = _minwhere(s1 == m_sc)
                cb = jnp.where(c0 < 16, c0, c1 + 16)
                lane = jnp.where(cb < 16, cb, cb - 16)
                hvec = jnp.where(cb < 16, h0, h1)
                hval = jnp.max(jnp.where(it16 == lane, hvec, 0))
                sel0 = (it16 == lane) & (cb < 16)
                sel1 = (it16 == lane) & (cb >= 16)
                h0n = jnp.where(sel0, h0 + 1, h0)
                h1n = jnp.where(sel1, h1 + 1, h1)

                _scat1(osc_v, r, m_sc)
                _scat1(olab_v, r, cb)
                crd = jnp.minimum(it16, jnp.int32(3))
                bxv = plsc.load_gather(
                    kbx_v, [(cb * 4 + crd) * _KCAP + hval])
                plsc.store_scatter(obf_v, [r * 4 + crd], bxv, mask=it16 < 4)
                fidx = plsc.load_gather(kid_v, [_bfull(cb * _KCAP + hval,
                                                       jnp.int32)])
                gidx = fidx + b * _N
                ridx = gidx // 8
                sub = gidx % 8
                m0 = it16 == 0
                _scat1(sub_v, r, sub)
                pa = _bfull(jnp.minimum(r, 127), jnp.int32)
                pb = _bfull(jnp.clip(r - 128, 0, 127), jnp.int32)
                pc = _bfull(jnp.clip(r - 256, 0, _KCAP - 257), jnp.int32)
                plsc.store_scatter(idxa_v, [pa], ridx, mask=m0 & (r < 128))
                plsc.store_scatter(idxb_v, [pb], ridx,
                                   mask=m0 & (r >= 128) & (r < 256))
                plsc.store_scatter(idxc_v, [pc], ridx, mask=m0 & (r >= 256))
                return r + 1, h0n, h1n

            go = m_sc > _NEG
            r2, h0n, h1n = lax.cond(go, live, lambda op: op, (r, h0, h1))
            return r2, h0n, h1n, go & (r2 < _MAXD)

        rf, _, _, _ = lax.while_loop(
            mg_cond, mg_body, (jnp.int32(0), zeros, zeros, True))

        g1 = pltpu.async_copy(lc_hbm.at[idxa_v], lcr_v.at[pl.ds(0, 128)], sem)
        g2 = pltpu.async_copy(
            lc_hbm.at[idxb_v], lcr_v.at[pl.ds(128, 128)], sem)
        g3 = pltpu.async_copy(
            lc_hbm.at[idxc_v], lcr_v.at[pl.ds(256, _KCAP - 256)], sem)
        g1.wait()
        g2.wait()
        g3.wait()

        def lcl(r, _):
            rv = _bfull(r, jnp.int32)
            sub16 = plsc.load_gather(sub_v, [rv])
            row = plsc.load_gather(lcr_v, [rv, sub16 * 16 + it16])
            lm = jnp.max(row)
            _scat1(olsc_v, r, lm)
            _scat1(ollab_v, r, _minwhere(row == lm))
            return 0
        lax.fori_loop(0, rf, lcl, 0)

        pltpu.sync_copy(
            obf_v, obox_hbm.at[pl.ds(_al8(b * _KCAP * 4), _KCAP * 4)])
        pltpu.sync_copy(osc_v, osc_hbm.at[pl.ds(_al8(b * _KCAP), _KCAP)])
        pltpu.sync_copy(olab_v, olab_hbm.at[pl.ds(_al8(b * _KCAP), _KCAP)])
        pltpu.sync_copy(olsc_v, olsc_hbm.at[pl.ds(_al8(b * _KCAP), _KCAP)])
        pltpu.sync_copy(ollab_v, ollab_hbm.at[pl.ds(_al8(b * _KCAP), _KCAP)])


_phase2 = pl.kernel(
    _merge_body,
    out_type=(
        jax.ShapeDtypeStruct((_B * _KCAP * 4,), jnp.float32),
        jax.ShapeDtypeStruct((_B * _KCAP,), jnp.float32),
        jax.ShapeDtypeStruct((_B * _KCAP,), jnp.int32),
        jax.ShapeDtypeStruct((_B * _KCAP,), jnp.float32),
        jax.ShapeDtypeStruct((_B * _KCAP,), jnp.int32),
    ),
    mesh=_mesh,
    compiler_params=_params,
    scratch_types=[
        pltpu.VMEM((_C * _KCAP,), jnp.float32),
        pltpu.VMEM((_C * _KCAP,), jnp.int32),
        pltpu.VMEM((_C * 4 * _KCAP,), jnp.float32),
        pltpu.VMEM((_C * 16,), jnp.int32),
        pltpu.VMEM((_KCAP * 4,), jnp.float32),
        pltpu.VMEM((_KCAP,), jnp.float32),
        pltpu.VMEM((_KCAP,), jnp.int32),
        pltpu.VMEM((_KCAP,), jnp.float32),
        pltpu.VMEM((_KCAP,), jnp.int32),
        pltpu.VMEM((128,), jnp.int32),
        pltpu.VMEM((128,), jnp.int32),
        pltpu.VMEM((_KCAP - 256,), jnp.int32),
        pltpu.VMEM((_KCAP,), jnp.int32),
        pltpu.VMEM((_KCAP, _LCROW), jnp.float32),
        pltpu.SemaphoreType.DMA,
    ],
)


@jax.jit
def kernel(boxes, classification, l_classification):
    cls_t = jnp.transpose(classification, (0, 2, 1)).reshape(-1)
    bx_t = jnp.transpose(boxes, (0, 2, 1)).reshape(-1)
    lc = jnp.pad(
        l_classification, ((0, 0), (0, 0), (0, 16 - _LC)),
        constant_values=-np.inf,
    ).reshape(_B * _N * 16 // _LCROW, _LCROW)
    ksc, kid, kbx, kcnt = _phase1(cls_t, bx_t)
    obox, osc, olab, olsc, ollab = _phase2(ksc, kid, kbx, kcnt, lc)
    return (
        obox.reshape(_B, _KCAP, 4)[:, :_MAXD],
        osc.reshape(_B, _KCAP)[:, :_MAXD],
        olab.reshape(_B, _KCAP)[:, :_MAXD],
        olsc.reshape(_B, _KCAP)[:, :_MAXD],
        ollab.reshape(_B, _KCAP)[:, :_MAXD],
    )


# speedup vs baseline: 1.1346x; 1.0376x over previous
import jax
import jax.numpy as jnp
import numpy as np
from jax import lax
from jax.experimental import pallas as pl
from jax.experimental.pallas import tpu as pltpu
from jax.experimental.pallas import tpu_sc as plsc

_B, _N, _C, _LC = 4, 20000, 20, 10
_TH = 0.05
_MAXD = 300
_KCAP = 304
_NG = _N // 16
_NSG = (_NG + 15) // 16
_L1P = _NSG * 16
_L2P = 80
_NT = _B * _C
_NW = 32
_NEG = float("-inf")
_LCROW = 128

_mesh = plsc.VectorSubcoreMesh(
    core_axis_name="c", subcore_axis_name="s", num_cores=2, num_subcores=16
)
_params = pltpu.CompilerParams(needs_layout_passes=False)


_SCPAD = _L1P * 16
_L1PAD = _L2P * 16
_BIG = 3.0e38


def _iota():
    return lax.iota(jnp.int32, 16)


def _minwhere(mask):
    return plsc.all_reduce_ffs(mask)[0]


def _bfull(x, dtype=None):
    v = jnp.full((16,), x)
    return v if dtype is None else v.astype(dtype)


def _al8(x):
    return pl.multiple_of(x, 8)


def _vmaxsplat(v):
    i = _iota()
    for sh in (8, 4, 2, 1):
        v = jnp.maximum(v, jnp.take(v, i ^ sh))
    return v


def _scat1(ref, pos, val, extra_mask=None):
    m = _iota() == 0
    if extra_mask is not None:
        m = m & extra_mask
    plsc.store_scatter(ref, [_bfull(pos, jnp.int32)], _bfull(val), mask=m)


def _nms_body(cls_hbm, bx_hbm, ksc_hbm, kid_hbm, kbx_hbm, kcnt_hbm,
              sc_v, x1_v, y1_v, x2_v, y2_v, l1_v,
              kx1_v, ky1_v, kx2_v, ky2_v, kar_v, ksl_v, kil_v, cnt_v, sem):
    wid = lax.axis_index("s") * 2 + lax.axis_index("c")
    it16 = _iota()

    def do_task(t):
        b = t // _C
        cps = [
            pltpu.async_copy(
                cls_hbm.at[pl.ds(_al8(t * _N), _N)],
                sc_v.at[pl.ds(0, _N)], sem),
            pltpu.async_copy(
                bx_hbm.at[pl.ds(_al8(b * 4 * _N), _N)], x1_v, sem),
            pltpu.async_copy(
                bx_hbm.at[pl.ds(_al8((b * 4 + 1) * _N), _N)], y1_v, sem),
            pltpu.async_copy(
                bx_hbm.at[pl.ds(_al8((b * 4 + 2) * _N), _N)], x2_v, sem),
            pltpu.async_copy(
                bx_hbm.at[pl.ds(_al8((b * 4 + 3) * _N), _N)], y2_v, sem),
        ]
        for k in range((_SCPAD - _N) // 16):
            sc_v[pl.ds(_N + k * 16, 16)] = _bfull(_NEG)
        l1_v[pl.ds(_L1P, 16)] = _bfull(_NEG)

        def initk(k, _):
            for ref in (kx1_v, ky1_v, kx2_v, ky2_v, kar_v):
                ref[pl.ds(k * 16, 16)] = _bfull(jnp.float32(_BIG))
            return 0
        lax.fori_loop(0, _KCAP // 16, initk, 0)

        for cp in cps:
            cp.wait()

        def build(cidx):
            base = (cidx * 16 + it16) * 16
            mx = _bfull(_NEG)
            for j in range(16):
                mx = jnp.maximum(mx, plsc.load_gather(sc_v, [base + j]))
            mx = jnp.where(mx > jnp.float32(_TH), mx, _NEG)
            l1_v[pl.ds(cidx * 16, 16)] = mx
        plsc.parallel_loop(0, _NSG, 1, unroll=2)(build)

        l2init = []
        for c in range(_L2P // 16):
            base = (c * 16 + it16) * 16
            mx = _bfull(_NEG)
            for j in range(16):
                mx = jnp.maximum(mx, plsc.load_gather(l1_v, [base + j]))
            l2init.append(mx)
        l2init = tuple(l2init)

        def ex_cond(carry):
            return carry[3]

        def ex_body(carry):
            kc, kcv, it, _, l2r = carry
            mv = l2r[0]
            for k in range(1, _L2P // 16):
                mv = jnp.maximum(mv, l2r[k])
            msp = _vmaxsplat(mv)
            m_ok = msp[0] > _NEG

            def live(op):
                kc, kcv, l2r = op
                g2v = _bfull(jnp.int32(9 * _L2P))
                for k in range(_L2P // 16):
                    nkv = plsc.all_reduce_ffs(l2r[k] == msp)
                    g2v = jnp.minimum(
                        g2v, jnp.where(nkv < 16, k * 16 + nkv, 9 * _L2P))
                l1g = plsc.load_gather(l1_v, [g2v * 16 + it16])
                n1v = plsc.all_reduce_ffs(l1g == msp)
                gv = g2v * 16 + n1v
                sg = plsc.load_gather(sc_v, [gv * 16 + it16])
                lanev = plsc.all_reduce_ffs(sg == msp)
                fv = gv * 16 + lanev
                cx1 = plsc.load_gather(x1_v, [fv])
                cy1 = plsc.load_gather(y1_v, [fv])
                cx2 = plsc.load_gather(x2_v, [fv])
                cy2 = plsc.load_gather(y2_v, [fv])
                ca = (cx2 - cx1) * (cy2 - cy1)

                nblk = (kc + 15) // 16

                def iou_body(j, acc):
                    a1 = kx1_v[pl.ds(j * 16, 16)]
                    b1 = ky1_v[pl.ds(j * 16, 16)]
                    a2 = kx2_v[pl.ds(j * 16, 16)]
                    b2 = ky2_v[pl.ds(j * 16, 16)]
                    ar = kar_v[pl.ds(j * 16, 16)]
                    iw = jnp.minimum(cx2, a2) - jnp.maximum(cx1, a1)
                    ih = jnp.minimum(cy2, b2) - jnp.maximum(cy1, b1)
                    inter = jnp.maximum(iw, 0.0) * jnp.maximum(ih, 0.0)
                    un = jnp.maximum(ar + ca - inter, jnp.float32(1e-8))
                    return acc | (inter > jnp.float32(0.5) * un)

                acc = plsc.parallel_loop(
                    0, nblk, 1, unroll=2, carry=it16 < 0)(iou_body)
                popc = plsc.all_reduce_population_count(acc)
                supv = popc > 0

                keepm = (it16 == 0) & (~supv)
                plsc.store_scatter(kx1_v, [kcv], cx1, mask=keepm)
                plsc.store_scatter(ky1_v, [kcv], cy1, mask=keepm)
                plsc.store_scatter(kx2_v, [kcv], cx2, mask=keepm)
                plsc.store_scatter(ky2_v, [kcv], cy2, mask=keepm)
                plsc.store_scatter(kar_v, [kcv], ca, mask=keepm)
                plsc.store_scatter(ksl_v, [kcv], msp, mask=keepm)
                plsc.store_scatter(kil_v, [kcv], fv, mask=keepm)
                kcv2 = jnp.where(supv, kcv, kcv + 1)
                kc2 = kc + jnp.where(popc[0] > 0, 0, 1)

                sg2 = jnp.where(it16 == lanev, _NEG, sg)
                plsc.store_scatter(sc_v, [gv * 16 + it16], sg2)
                gmr = _vmaxsplat(sg2)
                gm = jnp.where(gmr > jnp.float32(_TH), gmr, _NEG)
                l1g2 = jnp.where(it16 == n1v, gm, l1g)
                plsc.store_scatter(l1_v, [g2v * 16 + it16], l1g2)
                sgm = _vmaxsplat(l1g2)
                kvreg = g2v >> 4
                klane = g2v & jnp.int32(15)
                l2n = tuple(
                    jnp.where((kvreg == k) & (it16 == klane), sgm, l2r[k])
                    for k in range(_L2P // 16))
                return kc2, kcv2, l2n

            kc2, kcv2, l2n = lax.cond(
                m_ok, live, lambda op: op, (kc, kcv, l2r))
            go = m_ok & (kc2 < _MAXD) & (it + 1 < _N)
            return kc2, kcv2, it + 1, go, l2n

        kcf, _, _, _, _ = lax.while_loop(
            ex_cond, ex_body,
            (jnp.int32(0), _bfull(jnp.int32(0)), jnp.int32(0), True, l2init))

        cnt_v[pl.ds(0, 16)] = _bfull(kcf, jnp.int32)
        pltpu.sync_copy(cnt_v, kcnt_hbm.at[pl.ds(_al8(t * 16), 16)])
        pltpu.sync_copy(ksl_v, ksc_hbm.at[pl.ds(_al8(t * _KCAP), _KCAP)])
        pltpu.sync_copy(kil_v, kid_hbm.at[pl.ds(_al8(t * _KCAP), _KCAP)])
        for j, ref in enumerate((kx1_v, ky1_v, kx2_v, ky2_v)):
            pltpu.sync_copy(
                ref, kbx_hbm.at[pl.ds(_al8((t * 4 + j) * _KCAP), _KCAP)])

    def tloop(i, _):
        t = wid + _NW * i

        @pl.when(t < _NT)
        def _():
            do_task(t)

        return 0

    lax.fori_loop(0, (_NT + _NW - 1) // _NW, tloop, 0)


_phase1 = pl.kernel(
    _nms_body,
    out_type=(
        jax.ShapeDtypeStruct((_NT * _KCAP,), jnp.float32),
        jax.ShapeDtypeStruct((_NT * _KCAP,), jnp.int32),
        jax.ShapeDtypeStruct((_NT * 4 * _KCAP,), jnp.float32),
        jax.ShapeDtypeStruct((_NT * 16,), jnp.int32),
    ),
    mesh=_mesh,
    compiler_params=_params,
    scratch_types=[
        pltpu.VMEM((_SCPAD,), jnp.float32),
        pltpu.VMEM((_N,), jnp.float32),
        pltpu.VMEM((_N,), jnp.float32),
        pltpu.VMEM((_N,), jnp.float32),
        pltpu.VMEM((_N,), jnp.float32),
        pltpu.VMEM((_L1PAD,), jnp.float32),
        pltpu.VMEM((_KCAP,), jnp.float32),
        pltpu.VMEM((_KCAP,), jnp.float32),
        pltpu.VMEM((_KCAP,), jnp.float32),
        pltpu.VMEM((_KCAP,), jnp.float32),
        pltpu.VMEM((_KCAP,), jnp.float32),
        pltpu.VMEM((_KCAP,), jnp.float32),
        pltpu.VMEM((_KCAP,), jnp.int32),
        pltpu.VMEM((16,), jnp.int32),
        pltpu.SemaphoreType.DMA,
    ],
)


def _merge_body(ksc_hbm, kid_hbm, kbx_hbm, kcnt_hbm, lc_hbm,
                obox_hbm, osc_hbm, olab_hbm, olsc_hbm, ollab_hbm,
                ksc_v, kid_v, kbx_v, kcnt_v,
                obf_v, osc_v, olab_v, olsc_v, ollab_v,
                idxa_v, idxb_v, idxc_v, sub_v, lcr_v, sem):
    wid = lax.axis_index("s") * 2 + lax.axis_index("c")
    it16 = _iota()

    @pl.when(wid < _B)
    def _():
        b = wid
        cps = [
            pltpu.async_copy(
                ksc_hbm.at[pl.ds(_al8(b * _C * _KCAP), _C * _KCAP)],
                ksc_v, sem),
            pltpu.async_copy(
                kid_hbm.at[pl.ds(_al8(b * _C * _KCAP), _C * _KCAP)],
                kid_v, sem),
            pltpu.async_copy(
                kbx_hbm.at[pl.ds(_al8(b * _C * 4 * _KCAP), _C * 4 * _KCAP)],
                kbx_v, sem),
            pltpu.async_copy(
                kcnt_hbm.at[pl.ds(_al8(b * _C * 16), _C * 16)], kcnt_v, sem),
        ]
        for cp in cps:
            cp.wait()

        def initf(k, _):
            osc_v[pl.ds(k * 16, 16)] = _bfull(jnp.float32(-1.0))
            olsc_v[pl.ds(k * 16, 16)] = _bfull(jnp.float32(-1.0))
            olab_v[pl.ds(k * 16, 16)] = _bfull(jnp.int32(-1))
            ollab_v[pl.ds(k * 16, 16)] = _bfull(jnp.int32(-1))
            sub_v[pl.ds(k * 16, 16)] = _bfull(jnp.int32(0))
            return 0
        lax.fori_loop(0, _KCAP // 16, initf, 0)

        def initb(k, _):
            obf_v[pl.ds(k * 16, 16)] = _bfull(jnp.float32(-1.0))
            return 0
        lax.fori_loop(0, (_KCAP * 4) // 16, initb, 0)

        def initi(k, _):
            @pl.when(k < 8)
            def _():
                idxa_v[pl.ds(k * 16, 16)] = _bfull(jnp.int32(0))
                idxb_v[pl.ds(k * 16, 16)] = _bfull(jnp.int32(0))

            @pl.when(k < 3)
            def _():
                idxc_v[pl.ds(k * 16, 16)] = _bfull(jnp.int32(0))

            return 0
        lax.fori_loop(0, 8, initi, 0)

        zeros = _bfull(jnp.int32(0))
        row1 = jnp.minimum(it16 + 16, jnp.int32(_C - 1))
        cnt0 = plsc.load_gather(kcnt_v, [it16 * 16])
        cnt1 = plsc.load_gather(kcnt_v, [row1 * 16])
        lane_ok1 = it16 < (_C - 16)

        s0i = plsc.load_gather(ksc_v, [it16 * _KCAP])
        s0i = jnp.where(cnt0 > 0, s0i, _NEG)
        s1i = plsc.load_gather(ksc_v, [row1 * _KCAP])
        s1i = jnp.where(lane_ok1 & (cnt1 > 0), s1i, _NEG)

        def mg_cond(carry):
            return carry[3]

        def mg_body(carry):
            r, h0, h1, _, s0, s1 = carry
            msp = _vmaxsplat(jnp.maximum(s0, s1))
            m_ok = msp[0] > _NEG

            def live(op):
                r, h0, h1, s0, s1 = op
                c0v = plsc.all_reduce_ffs(s0 == msp)
                c1v = plsc.all_reduce_ffs(s1 == msp)
                cbv = jnp.where(c0v < 16, c0v, c1v + 16)
                lanev = cbv & jnp.int32(15)
                in0 = cbv < 16
                hval = jnp.take(jnp.where(in0, h0, h1), lanev)
                sel0 = (it16 == lanev) & in0
                sel1 = (it16 == lanev) & (~in0)
                h0n = jnp.where(sel0, h0 + 1, h0)
                h1n = jnp.where(sel1, h1 + 1, h1)

                m0 = it16 == 0
                posv = _bfull(r, jnp.int32)
                plsc.store_scatter(osc_v, [posv], msp, mask=m0)
                plsc.store_scatter(olab_v, [posv], cbv, mask=m0)
                crd = jnp.minimum(it16, jnp.int32(3))
                bxv = plsc.load_gather(
                    kbx_v, [(cbv * 4 + crd) * _KCAP + hval])
                plsc.store_scatter(obf_v, [r * 4 + crd], bxv, mask=it16 < 4)
                fidx = plsc.load_gather(kid_v, [cbv * _KCAP + hval])
                gidx = fidx + b * _N
                ridx = gidx >> 3
                sub = gidx & jnp.int32(7)
                plsc.store_scatter(sub_v, [posv], sub, mask=m0)
                pa = _bfull(jnp.minimum(r, 127), jnp.int32)
                pb = _bfull(jnp.clip(r - 128, 0, 127), jnp.int32)
                pc = _bfull(jnp.clip(r - 256, 0, _KCAP - 257), jnp.int32)
                plsc.store_scatter(idxa_v, [pa], ridx, mask=m0 & (r < 128))
                plsc.store_scatter(idxb_v, [pb], ridx,
                                   mask=m0 & (r >= 128) & (r < 256))
                plsc.store_scatter(idxc_v, [pc], ridx, mask=m0 & (r >= 256))

                hn = hval + 1
                cntb = plsc.load_gather(kcnt_v, [cbv * 16])
                news = plsc.load_gather(
                    ksc_v, [cbv * _KCAP + jnp.minimum(hn, _KCAP - 1)])
                news = jnp.where(hn < cntb, news, _NEG)
                s0n = jnp.where(sel0, news, s0)
                s1n = jnp.where(sel1, news, s1)
                return r + 1, h0n, h1n, s0n, s1n

            r2, h0n, h1n, s0n, s1n = lax.cond(
                m_ok, live, lambda op: op, (r, h0, h1, s0, s1))
            return r2, h0n, h1n, m_ok & (r2 < _MAXD), s0n, s1n

        rf, _, _, _, _, _ = lax.while_loop(
            mg_cond, mg_body, (jnp.int32(0), zeros, zeros, True, s0i, s1i))

        g1 = pltpu.async_copy(lc_hbm.at[idxa_v], lcr_v.at[pl.ds(0, 128)], sem)
        g2 = pltpu.async_copy(
            lc_hbm.at[idxb_v], lcr_v.at[pl.ds(128, 128)], sem)
        g3 = pltpu.async_copy(
            lc_hbm.at[idxc_v], lcr_v.at[pl.ds(256, _KCAP - 256)], sem)
        g1.wait()
        g2.wait()
        g3.wait()

        def lcl(r, _):
            rv = _bfull(r, jnp.int32)
            sub16 = plsc.load_gather(sub_v, [rv])
            row = plsc.load_gather(lcr_v, [rv, sub16 * 16 + it16])
            lm = jnp.max(row)
            _scat1(olsc_v, r, lm)
            _scat1(ollab_v, r, _minwhere(row == lm))
            return 0
        lax.fori_loop(0, rf, lcl, 0)

        pltpu.sync_copy(
            obf_v, obox_hbm.at[pl.ds(_al8(b * _KCAP * 4), _KCAP * 4)])
        pltpu.sync_copy(osc_v, osc_hbm.at[pl.ds(_al8(b * _KCAP), _KCAP)])
        pltpu.sync_copy(olab_v, olab_hbm.at[pl.ds(_al8(b * _KCAP), _KCAP)])
        pltpu.sync_copy(olsc_v, olsc_hbm.at[pl.ds(_al8(b * _KCAP), _KCAP)])
        pltpu.sync_copy(ollab_v, ollab_hbm.at[pl.ds(_al8(b * _KCAP), _KCAP)])


_phase2 = pl.kernel(
    _merge_body,
    out_type=(
        jax.ShapeDtypeStruct((_B * _KCAP * 4,), jnp.float32),
        jax.ShapeDtypeStruct((_B * _KCAP,), jnp.float32),
        jax.ShapeDtypeStruct((_B * _KCAP,), jnp.int32),
        jax.ShapeDtypeStruct((_B * _KCAP,), jnp.float32),
        jax.ShapeDtypeStruct((_B * _KCAP,), jnp.int32),
    ),
    mesh=_mesh,
    compiler_params=_params,
    scratch_types=[
        pltpu.VMEM((_C * _KCAP,), jnp.float32),
        pltpu.VMEM((_C * _KCAP,), jnp.int32),
        pltpu.VMEM((_C * 4 * _KCAP,), jnp.float32),
        pltpu.VMEM((_C * 16,), jnp.int32),
        pltpu.VMEM((_KCAP * 4,), jnp.float32),
        pltpu.VMEM((_KCAP,), jnp.float32),
        pltpu.VMEM((_KCAP,), jnp.int32),
        pltpu.VMEM((_KCAP,), jnp.float32),
        pltpu.VMEM((_KCAP,), jnp.int32),
        pltpu.VMEM((128,), jnp.int32),
        pltpu.VMEM((128,), jnp.int32),
        pltpu.VMEM((_KCAP - 256,), jnp.int32),
        pltpu.VMEM((_KCAP,), jnp.int32),
        pltpu.VMEM((_KCAP, _LCROW), jnp.float32),
        pltpu.SemaphoreType.DMA,
    ],
)


@jax.jit
def kernel(boxes, classification, l_classification):
    cls_t = jnp.transpose(classification, (0, 2, 1)).reshape(-1)
    bx_t = jnp.transpose(boxes, (0, 2, 1)).reshape(-1)
    lc = jnp.pad(
        l_classification, ((0, 0), (0, 0), (0, 16 - _LC)),
        constant_values=-np.inf,
    ).reshape(_B * _N * 16 // _LCROW, _LCROW)
    ksc, kid, kbx, kcnt = _phase1(cls_t, bx_t)
    obox, osc, olab, olsc, ollab = _phase2(ksc, kid, kbx, kcnt, lc)
    return (
        obox.reshape(_B, _KCAP, 4)[:, :_MAXD],
        osc.reshape(_B, _KCAP)[:, :_MAXD],
        olab.reshape(_B, _KCAP)[:, :_MAXD],
        olsc.reshape(_B, _KCAP)[:, :_MAXD],
        ollab.reshape(_B, _KCAP)[:, :_MAXD],
    )


# speedup vs baseline: 1.1623x; 1.0244x over previous
import jax
import jax.numpy as jnp
import numpy as np
from jax import lax
from jax.experimental import pallas as pl
from jax.experimental.pallas import tpu as pltpu
from jax.experimental.pallas import tpu_sc as plsc

_B, _N, _C, _LC = 4, 20000, 20, 10
_TH = 0.05
_MAXD = 300
_KCAP = 304
_NG = _N // 16
_NSG = (_NG + 15) // 16
_L1P = _NSG * 16
_L2P = 80
_NT = _B * _C
_NW = 32
_NEG = float("-inf")
_LCROW = 128

_mesh = plsc.VectorSubcoreMesh(
    core_axis_name="c", subcore_axis_name="s", num_cores=2, num_subcores=16
)
_params = pltpu.CompilerParams(needs_layout_passes=False)


_SCPAD = _L1P * 16
_L1PAD = _L2P * 16
_BIG = 3.0e38


def _iota():
    return lax.iota(jnp.int32, 16)


def _minwhere(mask):
    return plsc.all_reduce_ffs(mask)[0]


def _bfull(x, dtype=None):
    v = jnp.full((16,), x)
    return v if dtype is None else v.astype(dtype)


def _al8(x):
    return pl.multiple_of(x, 8)


def _vmaxsplat(v):
    i = _iota()
    for sh in (8, 4, 2, 1):
        v = jnp.maximum(v, jnp.take(v, i ^ sh))
    return v


def _scat1(ref, pos, val, extra_mask=None):
    m = _iota() == 0
    if extra_mask is not None:
        m = m & extra_mask
    plsc.store_scatter(ref, [_bfull(pos, jnp.int32)], _bfull(val), mask=m)


def _nms_body(cls_hbm, bx_hbm, ksc_hbm, kid_hbm, kbx_hbm, kcnt_hbm,
              sc_v, x1_v, y1_v, x2_v, y2_v, l1_v,
              kx1_v, ky1_v, kx2_v, ky2_v, kar_v, ksl_v, kil_v, cnt_v, sem):
    wid = lax.axis_index("s") * 2 + lax.axis_index("c")
    it16 = _iota()

    def do_task(t):
        b = t // _C
        cps = [
            pltpu.async_copy(
                cls_hbm.at[pl.ds(_al8(t * _N), _N)],
                sc_v.at[pl.ds(0, _N)], sem),
            pltpu.async_copy(
                bx_hbm.at[pl.ds(_al8(b * 4 * _N), _N)], x1_v, sem),
            pltpu.async_copy(
                bx_hbm.at[pl.ds(_al8((b * 4 + 1) * _N), _N)], y1_v, sem),
            pltpu.async_copy(
                bx_hbm.at[pl.ds(_al8((b * 4 + 2) * _N), _N)], x2_v, sem),
            pltpu.async_copy(
                bx_hbm.at[pl.ds(_al8((b * 4 + 3) * _N), _N)], y2_v, sem),
        ]
        for k in range((_SCPAD - _N) // 16):
            sc_v[pl.ds(_N + k * 16, 16)] = _bfull(_NEG)
        l1_v[pl.ds(_L1P, 16)] = _bfull(_NEG)

        def initk(k, _):
            for ref in (kx1_v, ky1_v, kx2_v, ky2_v, kar_v):
                ref[pl.ds(k * 16, 16)] = _bfull(jnp.float32(_BIG))
            return 0
        lax.fori_loop(0, _KCAP // 16, initk, 0)

        for cp in cps:
            cp.wait()

        def build(cidx):
            base = (cidx * 16 + it16) * 16
            mx = _bfull(_NEG)
            for j in range(16):
                mx = jnp.maximum(mx, plsc.load_gather(sc_v, [base + j]))
            mx = jnp.where(mx > jnp.float32(_TH), mx, _NEG)
            l1_v[pl.ds(cidx * 16, 16)] = mx
        plsc.parallel_loop(0, _NSG, 1, unroll=2)(build)

        l2init = []
        for c in range(_L2P // 16):
            base = (c * 16 + it16) * 16
            mx = _bfull(_NEG)
            for j in range(16):
                mx = jnp.maximum(mx, plsc.load_gather(l1_v, [base + j]))
            l2init.append(mx)
        l2init = tuple(l2init)

        def ex_cond(carry):
            return carry[3]

        def ex_body(carry):
            kc, kcv, it, _, l2r = carry
            mv = l2r[0]
            for k in range(1, _L2P // 16):
                mv = jnp.maximum(mv, l2r[k])
            msp = _vmaxsplat(mv)
            m_ok = msp[0] > _NEG

            def live(op):
                kc, kcv, l2r = op
                g2v = _bfull(jnp.int32(9 * _L2P))
                for k in range(_L2P // 16):
                    nkv = plsc.all_reduce_ffs(l2r[k] == msp)
                    g2v = jnp.minimum(
                        g2v, jnp.where(nkv < 16, k * 16 + nkv, 9 * _L2P))
                l1g = plsc.load_gather(l1_v, [g2v * 16 + it16])
                n1v = plsc.all_reduce_ffs(l1g == msp)
                gv = g2v * 16 + n1v
                sg = plsc.load_gather(sc_v, [gv * 16 + it16])
                lanev = plsc.all_reduce_ffs(sg == msp)
                fv = gv * 16 + lanev
                cx1 = plsc.load_gather(x1_v, [fv])
                cy1 = plsc.load_gather(y1_v, [fv])
                cx2 = plsc.load_gather(x2_v, [fv])
                cy2 = plsc.load_gather(y2_v, [fv])
                ca = (cx2 - cx1) * (cy2 - cy1)

                nblk = (kc + 15) // 16

                def iou_body(j, acc):
                    a1 = kx1_v[pl.ds(j * 16, 16)]
                    b1 = ky1_v[pl.ds(j * 16, 16)]
                    a2 = kx2_v[pl.ds(j * 16, 16)]
                    b2 = ky2_v[pl.ds(j * 16, 16)]
                    ar = kar_v[pl.ds(j * 16, 16)]
                    iw = jnp.minimum(cx2, a2) - jnp.maximum(cx1, a1)
                    ih = jnp.minimum(cy2, b2) - jnp.maximum(cy1, b1)
                    inter = jnp.maximum(iw, 0.0) * jnp.maximum(ih, 0.0)
                    un = jnp.maximum(ar + ca - inter, jnp.float32(1e-8))
                    return acc | (inter > jnp.float32(0.5) * un)

                acc = plsc.parallel_loop(
                    0, nblk, 1, unroll=2, carry=it16 < 0)(iou_body)
                popc = plsc.all_reduce_population_count(acc)
                supv = popc > 0

                keepm = (it16 == 0) & (~supv)
                plsc.store_scatter(kx1_v, [kcv], cx1, mask=keepm)
                plsc.store_scatter(ky1_v, [kcv], cy1, mask=keepm)
                plsc.store_scatter(kx2_v, [kcv], cx2, mask=keepm)
                plsc.store_scatter(ky2_v, [kcv], cy2, mask=keepm)
                plsc.store_scatter(kar_v, [kcv], ca, mask=keepm)
                plsc.store_scatter(ksl_v, [kcv], msp, mask=keepm)
                plsc.store_scatter(kil_v, [kcv], fv, mask=keepm)
                kcv2 = jnp.where(supv, kcv, kcv + 1)
                kc2 = kc + jnp.where(popc[0] > 0, 0, 1)

                sg2 = jnp.where(it16 == lanev, _NEG, sg)
                plsc.store_scatter(sc_v, [gv * 16 + it16], sg2)
                gmr = _vmaxsplat(sg2)
                gm = jnp.where(gmr > jnp.float32(_TH), gmr, _NEG)
                l1g2 = jnp.where(it16 == n1v, gm, l1g)
                plsc.store_scatter(l1_v, [g2v * 16 + it16], l1g2)
                sgm = _vmaxsplat(l1g2)
                kvreg = g2v >> 4
                klane = g2v & jnp.int32(15)
                l2n = tuple(
                    jnp.where((kvreg == k) & (it16 == klane), sgm, l2r[k])
                    for k in range(_L2P // 16))
                return kc2, kcv2, l2n

            kc2, kcv2, l2n = lax.cond(
                m_ok, live, lambda op: op, (kc, kcv, l2r))
            go = m_ok & (kc2 < _MAXD) & (it + 1 < _N)
            return kc2, kcv2, it + 1, go, l2n

        kcf, _, _, _, _ = lax.while_loop(
            ex_cond, ex_body,
            (jnp.int32(0), _bfull(jnp.int32(0)), jnp.int32(0), True, l2init))

        cnt_v[pl.ds(0, 16)] = _bfull(kcf, jnp.int32)
        ops = [
            pltpu.async_copy(cnt_v, kcnt_hbm.at[pl.ds(_al8(t * 16), 16)], sem),
            pltpu.async_copy(
                ksl_v, ksc_hbm.at[pl.ds(_al8(t * _KCAP), _KCAP)], sem),
            pltpu.async_copy(
                kil_v, kid_hbm.at[pl.ds(_al8(t * _KCAP), _KCAP)], sem),
        ]
        for j, ref in enumerate((kx1_v, ky1_v, kx2_v, ky2_v)):
            ops.append(pltpu.async_copy(
                ref, kbx_hbm.at[pl.ds(_al8((t * 4 + j) * _KCAP), _KCAP)], sem))
        for op in ops:
            op.wait()

    def tloop(i, _):
        t = wid + _NW * i

        @pl.when(t < _NT)
        def _():
            do_task(t)

        return 0

    lax.fori_loop(0, (_NT + _NW - 1) // _NW, tloop, 0)


_phase1 = pl.kernel(
    _nms_body,
    out_type=(
        jax.ShapeDtypeStruct((_NT * _KCAP,), jnp.float32),
        jax.ShapeDtypeStruct((_NT * _KCAP,), jnp.int32),
        jax.ShapeDtypeStruct((_NT * 4 * _KCAP,), jnp.float32),
        jax.ShapeDtypeStruct((_NT * 16,), jnp.int32),
    ),
    mesh=_mesh,
    compiler_params=_params,
    scratch_types=[
        pltpu.VMEM((_SCPAD,), jnp.float32),
        pltpu.VMEM((_N,), jnp.float32),
        pltpu.VMEM((_N,), jnp.float32),
        pltpu.VMEM((_N,), jnp.float32),
        pltpu.VMEM((_N,), jnp.float32),
        pltpu.VMEM((_L1PAD,), jnp.float32),
        pltpu.VMEM((_KCAP,), jnp.float32),
        pltpu.VMEM((_KCAP,), jnp.float32),
        pltpu.VMEM((_KCAP,), jnp.float32),
        pltpu.VMEM((_KCAP,), jnp.float32),
        pltpu.VMEM((_KCAP,), jnp.float32),
        pltpu.VMEM((_KCAP,), jnp.float32),
        pltpu.VMEM((_KCAP,), jnp.int32),
        pltpu.VMEM((16,), jnp.int32),
        pltpu.SemaphoreType.DMA,
    ],
)


def _merge_body(ksc_hbm, kid_hbm, kbx_hbm, kcnt_hbm, lc_hbm,
                obox_hbm, osc_hbm, olab_hbm, olsc_hbm, ollab_hbm,
                ksc_v, kid_v, kbx_v, kcnt_v,
                obf_v, osc_v, olab_v, olsc_v, ollab_v,
                idxa_v, idxb_v, idxc_v, sub_v, lcr_v, sem):
    wid = lax.axis_index("s") * 2 + lax.axis_index("c")
    it16 = _iota()

    @pl.when(wid < _B)
    def _():
        b = wid
        cps = [
            pltpu.async_copy(
                ksc_hbm.at[pl.ds(_al8(b * _C * _KCAP), _C * _KCAP)],
                ksc_v, sem),
            pltpu.async_copy(
                kid_hbm.at[pl.ds(_al8(b * _C * _KCAP), _C * _KCAP)],
                kid_v, sem),
            pltpu.async_copy(
                kbx_hbm.at[pl.ds(_al8(b * _C * 4 * _KCAP), _C * 4 * _KCAP)],
                kbx_v, sem),
            pltpu.async_copy(
                kcnt_hbm.at[pl.ds(_al8(b * _C * 16), _C * 16)], kcnt_v, sem),
        ]
        for cp in cps:
            cp.wait()

        def initf(k, _):
            osc_v[pl.ds(k * 16, 16)] = _bfull(jnp.float32(-1.0))
            olsc_v[pl.ds(k * 16, 16)] = _bfull(jnp.float32(-1.0))
            olab_v[pl.ds(k * 16, 16)] = _bfull(jnp.int32(-1))
            ollab_v[pl.ds(k * 16, 16)] = _bfull(jnp.int32(-1))
            sub_v[pl.ds(k * 16, 16)] = _bfull(jnp.int32(0))
            return 0
        lax.fori_loop(0, _KCAP // 16, initf, 0)

        def initb(k, _):
            obf_v[pl.ds(k * 16, 16)] = _bfull(jnp.float32(-1.0))
            return 0
        lax.fori_loop(0, (_KCAP * 4) // 16, initb, 0)

        def initi(k, _):
            @pl.when(k < 8)
            def _():
                idxa_v[pl.ds(k * 16, 16)] = _bfull(jnp.int32(0))
                idxb_v[pl.ds(k * 16, 16)] = _bfull(jnp.int32(0))

            @pl.when(k < 3)
            def _():
                idxc_v[pl.ds(k * 16, 16)] = _bfull(jnp.int32(0))

            return 0
        lax.fori_loop(0, 8, initi, 0)

        zeros = _bfull(jnp.int32(0))
        row1 = jnp.minimum(it16 + 16, jnp.int32(_C - 1))
        cnt0 = plsc.load_gather(kcnt_v, [it16 * 16])
        cnt1 = plsc.load_gather(kcnt_v, [row1 * 16])
        lane_ok1 = it16 < (_C - 16)

        s0i = plsc.load_gather(ksc_v, [it16 * _KCAP])
        s0i = jnp.where(cnt0 > 0, s0i, _NEG)
        s1i = plsc.load_gather(ksc_v, [row1 * _KCAP])
        s1i = jnp.where(lane_ok1 & (cnt1 > 0), s1i, _NEG)

        def mg_cond(carry):
            return carry[3]

        def mg_body(carry):
            r, h0, h1, _, s0, s1 = carry
            msp = _vmaxsplat(jnp.maximum(s0, s1))
            m_ok = msp[0] > _NEG

            def live(op):
                r, h0, h1, s0, s1 = op
                c0v = plsc.all_reduce_ffs(s0 == msp)
                c1v = plsc.all_reduce_ffs(s1 == msp)
                cbv = jnp.where(c0v < 16, c0v, c1v + 16)
                lanev = cbv & jnp.int32(15)
                in0 = cbv < 16
                hval = jnp.take(jnp.where(in0, h0, h1), lanev)
                sel0 = (it16 == lanev) & in0
                sel1 = (it16 == lanev) & (~in0)
                h0n = jnp.where(sel0, h0 + 1, h0)
                h1n = jnp.where(sel1, h1 + 1, h1)

                m0 = it16 == 0
                posv = _bfull(r, jnp.int32)
                plsc.store_scatter(osc_v, [posv], msp, mask=m0)
                plsc.store_scatter(olab_v, [posv], cbv, mask=m0)
                crd = jnp.minimum(it16, jnp.int32(3))
                bxv = plsc.load_gather(
                    kbx_v, [(cbv * 4 + crd) * _KCAP + hval])
                plsc.store_scatter(obf_v, [r * 4 + crd], bxv, mask=it16 < 4)
                fidx = plsc.load_gather(kid_v, [cbv * _KCAP + hval])
                gidx = fidx + b * _N
                ridx = gidx >> 3
                sub = gidx & jnp.int32(7)
                plsc.store_scatter(sub_v, [posv], sub, mask=m0)
                pa = _bfull(jnp.minimum(r, 127), jnp.int32)
                pb = _bfull(jnp.clip(r - 128, 0, 127), jnp.int32)
                pc = _bfull(jnp.clip(r - 256, 0, _KCAP - 257), jnp.int32)
                plsc.store_scatter(idxa_v, [pa], ridx, mask=m0 & (r < 128))
                plsc.store_scatter(idxb_v, [pb], ridx,
                                   mask=m0 & (r >= 128) & (r < 256))
                plsc.store_scatter(idxc_v, [pc], ridx, mask=m0 & (r >= 256))

                hn = hval + 1
                cntb = plsc.load_gather(kcnt_v, [cbv * 16])
                news = plsc.load_gather(
                    ksc_v, [cbv * _KCAP + jnp.minimum(hn, _KCAP - 1)])
                news = jnp.where(hn < cntb, news, _NEG)
                s0n = jnp.where(sel0, news, s0)
                s1n = jnp.where(sel1, news, s1)
                return r + 1, h0n, h1n, s0n, s1n

            r2, h0n, h1n, s0n, s1n = lax.cond(
                m_ok, live, lambda op: op, (r, h0, h1, s0, s1))
            return r2, h0n, h1n, m_ok & (r2 < _MAXD), s0n, s1n

        rf, _, _, _, _, _ = lax.while_loop(
            mg_cond, mg_body, (jnp.int32(0), zeros, zeros, True, s0i, s1i))

        g1 = pltpu.async_copy(lc_hbm.at[idxa_v], lcr_v.at[pl.ds(0, 128)], sem)
        g2 = pltpu.async_copy(
            lc_hbm.at[idxb_v], lcr_v.at[pl.ds(128, 128)], sem)
        g3 = pltpu.async_copy(
            lc_hbm.at[idxc_v], lcr_v.at[pl.ds(256, _KCAP - 256)], sem)
        g1.wait()
        g2.wait()
        g3.wait()

        def lcl(r):
            rv = _bfull(r, jnp.int32)
            sub16 = plsc.load_gather(sub_v, [rv])
            row = plsc.load_gather(lcr_v, [rv, sub16 * 16 + it16])
            lmv = _vmaxsplat(row)
            lav = plsc.all_reduce_ffs(row == lmv)
            m0 = it16 == 0
            plsc.store_scatter(olsc_v, [rv], lmv, mask=m0)
            plsc.store_scatter(ollab_v, [rv], lav, mask=m0)
        plsc.parallel_loop(0, rf, 1, unroll=2)(lcl)

        ops = [
            pltpu.async_copy(
                obf_v, obox_hbm.at[pl.ds(_al8(b * _KCAP * 4), _KCAP * 4)],
                sem),
            pltpu.async_copy(
                osc_v, osc_hbm.at[pl.ds(_al8(b * _KCAP), _KCAP)], sem),
            pltpu.async_copy(
                olab_v, olab_hbm.at[pl.ds(_al8(b * _KCAP), _KCAP)], sem),
            pltpu.async_copy(
                olsc_v, olsc_hbm.at[pl.ds(_al8(b * _KCAP), _KCAP)], sem),
            pltpu.async_copy(
                ollab_v, ollab_hbm.at[pl.ds(_al8(b * _KCAP), _KCAP)], sem),
        ]
        for op in ops:
            op.wait()


_phase2 = pl.kernel(
    _merge_body,
    out_type=(
        jax.ShapeDtypeStruct((_B * _KCAP * 4,), jnp.float32),
        jax.ShapeDtypeStruct((_B * _KCAP,), jnp.float32),
        jax.ShapeDtypeStruct((_B * _KCAP,), jnp.int32),
        jax.ShapeDtypeStruct((_B * _KCAP,), jnp.float32),
        jax.ShapeDtypeStruct((_B * _KCAP,), jnp.int32),
    ),
    mesh=_mesh,
    compiler_params=_params,
    scratch_types=[
        pltpu.VMEM((_C * _KCAP,), jnp.float32),
        pltpu.VMEM((_C * _KCAP,), jnp.int32),
        pltpu.VMEM((_C * 4 * _KCAP,), jnp.float32),
        pltpu.VMEM((_C * 16,), jnp.int32),
        pltpu.VMEM((_KCAP * 4,), jnp.float32),
        pltpu.VMEM((_KCAP,), jnp.float32),
        pltpu.VMEM((_KCAP,), jnp.int32),
        pltpu.VMEM((_KCAP,), jnp.float32),
        pltpu.VMEM((_KCAP,), jnp.int32),
        pltpu.VMEM((128,), jnp.int32),
        pltpu.VMEM((128,), jnp.int32),
        pltpu.VMEM((_KCAP - 256,), jnp.int32),
        pltpu.VMEM((_KCAP,), jnp.int32),
        pltpu.VMEM((_KCAP, _LCROW), jnp.float32),
        pltpu.SemaphoreType.DMA,
    ],
)


@jax.jit
def kernel(boxes, classification, l_classification):
    cls_t = jnp.transpose(classification, (0, 2, 1)).reshape(-1)
    bx_t = jnp.transpose(boxes, (0, 2, 1)).reshape(-1)
    lc = jnp.pad(
        l_classification, ((0, 0), (0, 0), (0, 16 - _LC)),
        constant_values=-np.inf,
    ).reshape(_B * _N * 16 // _LCROW, _LCROW)
    ksc, kid, kbx, kcnt = _phase1(cls_t, bx_t)
    obox, osc, olab, olsc, ollab = _phase2(ksc, kid, kbx, kcnt, lc)
    return (
        obox.reshape(_B, _KCAP, 4)[:, :_MAXD],
        osc.reshape(_B, _KCAP)[:, :_MAXD],
        olab.reshape(_B, _KCAP)[:, :_MAXD],
        olsc.reshape(_B, _KCAP)[:, :_MAXD],
        ollab.reshape(_B, _KCAP)[:, :_MAXD],
    )


# speedup vs baseline: 1.2086x; 1.0398x over previous
import jax
import jax.numpy as jnp
import numpy as np
from jax import lax
from jax.experimental import pallas as pl
from jax.experimental.pallas import tpu as pltpu
from jax.experimental.pallas import tpu_sc as plsc

_B, _N, _C, _LC = 4, 20000, 20, 10
_TH = 0.05
_MAXD = 300
_KCAP = 304
_NG = _N // 16
_NSG = (_NG + 15) // 16
_L1P = _NSG * 16
_L2P = 80
_NT = _B * _C
_NW = 32
_NEG = float("-inf")
_LCROW = 128

_mesh = plsc.VectorSubcoreMesh(
    core_axis_name="c", subcore_axis_name="s", num_cores=2, num_subcores=16
)
_params = pltpu.CompilerParams(needs_layout_passes=False)


_SCPAD = _L1P * 16
_L1PAD = _L2P * 16
_BIG = 3.0e38


def _iota():
    return lax.iota(jnp.int32, 16)


def _minwhere(mask):
    return plsc.all_reduce_ffs(mask)[0]


def _bfull(x, dtype=None):
    v = jnp.full((16,), x)
    return v if dtype is None else v.astype(dtype)


def _al8(x):
    return pl.multiple_of(x, 8)


def _vmaxsplat(v):
    i = _iota()
    for sh in (8, 4, 2, 1):
        v = jnp.maximum(v, jnp.take(v, i ^ sh))
    return v


def _scat1(ref, pos, val, extra_mask=None):
    m = _iota() == 0
    if extra_mask is not None:
        m = m & extra_mask
    plsc.store_scatter(ref, [_bfull(pos, jnp.int32)], _bfull(val), mask=m)


def _nms_body(cls_hbm, bx_hbm, ksc_hbm, kid_hbm, kbx_hbm, kcnt_hbm,
              sc_v, x1_v, y1_v, x2_v, y2_v, l1_v,
              kx1_v, ky1_v, kx2_v, ky2_v, kar_v, ksl_v, kil_v, cnt_v, sem):
    wid = lax.axis_index("s") * 2 + lax.axis_index("c")
    it16 = _iota()

    def do_task(t):
        b = t // _C
        cps = [
            pltpu.async_copy(
                cls_hbm.at[pl.ds(_al8(t * _N), _N)],
                sc_v.at[pl.ds(0, _N)], sem),
            pltpu.async_copy(
                bx_hbm.at[pl.ds(_al8(b * 4 * _N), _N)], x1_v, sem),
            pltpu.async_copy(
                bx_hbm.at[pl.ds(_al8((b * 4 + 1) * _N), _N)], y1_v, sem),
            pltpu.async_copy(
                bx_hbm.at[pl.ds(_al8((b * 4 + 2) * _N), _N)], x2_v, sem),
            pltpu.async_copy(
                bx_hbm.at[pl.ds(_al8((b * 4 + 3) * _N), _N)], y2_v, sem),
        ]
        for k in range((_SCPAD - _N) // 16):
            sc_v[pl.ds(_N + k * 16, 16)] = _bfull(_NEG)
        l1_v[pl.ds(_L1P, 16)] = _bfull(_NEG)

        def initk(k, _):
            for ref in (kx1_v, ky1_v, kx2_v, ky2_v, kar_v):
                ref[pl.ds(k * 16, 16)] = _bfull(jnp.float32(_BIG))
            return 0
        lax.fori_loop(0, _KCAP // 16, initk, 0)

        for cp in cps:
            cp.wait()

        def build(cidx):
            base = (cidx * 16 + it16) * 16
            mx = _bfull(_NEG)
            for j in range(16):
                mx = jnp.maximum(mx, plsc.load_gather(sc_v, [base + j]))
            mx = jnp.where(mx > jnp.float32(_TH), mx, _NEG)
            l1_v[pl.ds(cidx * 16, 16)] = mx
        plsc.parallel_loop(0, _NSG, 1, unroll=2)(build)

        l2init = []
        for c in range(_L2P // 16):
            base = (c * 16 + it16) * 16
            mx = _bfull(_NEG)
            for j in range(16):
                mx = jnp.maximum(mx, plsc.load_gather(l1_v, [base + j]))
            l2init.append(mx)
        l2init = tuple(l2init)

        def ex_cond(carry):
            return carry[3]

        def ex_body(carry):
            kc, kcv, it, _, l2r = carry
            mv = l2r[0]
            for k in range(1, _L2P // 16):
                mv = jnp.maximum(mv, l2r[k])
            msp = _vmaxsplat(mv)
            mokv = msp > _NEG
            m_ok = msp[0] > _NEG

            g2v = _bfull(jnp.int32(9 * _L2P))
            for k in range(_L2P // 16):
                nkv = plsc.all_reduce_ffs(l2r[k] == msp)
                g2v = jnp.minimum(
                    g2v, jnp.where(nkv < 16, k * 16 + nkv, 9 * _L2P))
            g2v = jnp.minimum(g2v, jnp.int32(_NSG - 1))
            l1g = plsc.load_gather(l1_v, [g2v * 16 + it16])
            n1v = jnp.minimum(plsc.all_reduce_ffs(l1g == msp), 15)
            gv = g2v * 16 + n1v
            sg = plsc.load_gather(sc_v, [gv * 16 + it16])
            lanev = jnp.minimum(plsc.all_reduce_ffs(sg == msp), 15)
            fv = gv * 16 + lanev
            cx1 = plsc.load_gather(x1_v, [fv])
            cy1 = plsc.load_gather(y1_v, [fv])
            cx2 = plsc.load_gather(x2_v, [fv])
            cy2 = plsc.load_gather(y2_v, [fv])
            ca = (cx2 - cx1) * (cy2 - cy1)

            nblk = (kc + 15) // 16

            def iou_body(j, acc):
                a1 = kx1_v[pl.ds(j * 16, 16)]
                b1 = ky1_v[pl.ds(j * 16, 16)]
                a2 = kx2_v[pl.ds(j * 16, 16)]
                b2 = ky2_v[pl.ds(j * 16, 16)]
                ar = kar_v[pl.ds(j * 16, 16)]
                iw = jnp.minimum(cx2, a2) - jnp.maximum(cx1, a1)
                ih = jnp.minimum(cy2, b2) - jnp.maximum(cy1, b1)
                inter = jnp.maximum(iw, 0.0) * jnp.maximum(ih, 0.0)
                un = jnp.maximum(ar + ca - inter, jnp.float32(1e-8))
                return acc | (inter > jnp.float32(0.5) * un)

            acc = plsc.parallel_loop(
                0, nblk, 1, unroll=2, carry=it16 < 0)(iou_body)
            popc = plsc.all_reduce_population_count(acc)
            supv = popc > 0

            keepm = (it16 == 0) & (~supv) & mokv
            plsc.store_scatter(kx1_v, [kcv], cx1, mask=keepm)
            plsc.store_scatter(ky1_v, [kcv], cy1, mask=keepm)
            plsc.store_scatter(kx2_v, [kcv], cx2, mask=keepm)
            plsc.store_scatter(ky2_v, [kcv], cy2, mask=keepm)
            plsc.store_scatter(kar_v, [kcv], ca, mask=keepm)
            plsc.store_scatter(ksl_v, [kcv], msp, mask=keepm)
            plsc.store_scatter(kil_v, [kcv], fv, mask=keepm)
            keep1 = jnp.where(supv, 0, 1)
            kcv2 = kcv + jnp.where(mokv, keep1, 0)
            kc2 = kc + jnp.where(m_ok & (popc[0] == 0), 1, 0)

            sg2 = jnp.where(it16 == lanev, _NEG, sg)
            plsc.store_scatter(sc_v, [gv * 16 + it16], sg2, mask=mokv)
            gmr = _vmaxsplat(sg2)
            gm = jnp.where(gmr > jnp.float32(_TH), gmr, _NEG)
            l1g2 = jnp.where(it16 == n1v, gm, l1g)
            plsc.store_scatter(l1_v, [g2v * 16 + it16], l1g2, mask=mokv)
            sgm = _vmaxsplat(l1g2)
            kvreg = g2v >> 4
            klane = g2v & jnp.int32(15)
            l2n = tuple(
                jnp.where(mokv & (kvreg == k) & (it16 == klane), sgm, l2r[k])
                for k in range(_L2P // 16))

            go = m_ok & (kc2 < _MAXD) & (it + 1 < _N)
            return kc2, kcv2, it + 1, go, l2n

        kcf, _, _, _, _ = lax.while_loop(
            ex_cond, ex_body,
            (jnp.int32(0), _bfull(jnp.int32(0)), jnp.int32(0), True, l2init))

        cnt_v[pl.ds(0, 16)] = _bfull(kcf, jnp.int32)
        ops = [
            pltpu.async_copy(cnt_v, kcnt_hbm.at[pl.ds(_al8(t * 16), 16)], sem),
            pltpu.async_copy(
                ksl_v, ksc_hbm.at[pl.ds(_al8(t * _KCAP), _KCAP)], sem),
            pltpu.async_copy(
                kil_v, kid_hbm.at[pl.ds(_al8(t * _KCAP), _KCAP)], sem),
        ]
        for j, ref in enumerate((kx1_v, ky1_v, kx2_v, ky2_v)):
            ops.append(pltpu.async_copy(
                ref, kbx_hbm.at[pl.ds(_al8((t * 4 + j) * _KCAP), _KCAP)], sem))
        for op in ops:
            op.wait()

    def tloop(i, _):
        t = wid + _NW * i

        @pl.when(t < _NT)
        def _():
            do_task(t)

        return 0

    lax.fori_loop(0, (_NT + _NW - 1) // _NW, tloop, 0)


_phase1 = pl.kernel(
    _nms_body,
    out_type=(
        jax.ShapeDtypeStruct((_NT * _KCAP,), jnp.float32),
        jax.ShapeDtypeStruct((_NT * _KCAP,), jnp.int32),
        jax.ShapeDtypeStruct((_NT * 4 * _KCAP,), jnp.float32),
        jax.ShapeDtypeStruct((_NT * 16,), jnp.int32),
    ),
    mesh=_mesh,
    compiler_params=_params,
    scratch_types=[
        pltpu.VMEM((_SCPAD,), jnp.float32),
        pltpu.VMEM((_N,), jnp.float32),
        pltpu.VMEM((_N,), jnp.float32),
        pltpu.VMEM((_N,), jnp.float32),
        pltpu.VMEM((_N,), jnp.float32),
        pltpu.VMEM((_L1PAD,), jnp.float32),
        pltpu.VMEM((_KCAP,), jnp.float32),
        pltpu.VMEM((_KCAP,), jnp.float32),
        pltpu.VMEM((_KCAP,), jnp.float32),
        pltpu.VMEM((_KCAP,), jnp.float32),
        pltpu.VMEM((_KCAP,), jnp.float32),
        pltpu.VMEM((_KCAP,), jnp.float32),
        pltpu.VMEM((_KCAP,), jnp.int32),
        pltpu.VMEM((16,), jnp.int32),
        pltpu.SemaphoreType.DMA,
    ],
)


def _merge_body(ksc_hbm, kid_hbm, kbx_hbm, kcnt_hbm, lc_hbm,
                obox_hbm, osc_hbm, olab_hbm, olsc_hbm, ollab_hbm,
                ksc_v, kid_v, kbx_v, kcnt_v,
                obf_v, osc_v, olab_v, olsc_v, ollab_v,
                idxa_v, idxb_v, idxc_v, sub_v, lcr_v, sem):
    wid = lax.axis_index("s") * 2 + lax.axis_index("c")
    it16 = _iota()

    @pl.when(wid < _B)
    def _():
        b = wid
        cps = [
            pltpu.async_copy(
                ksc_hbm.at[pl.ds(_al8(b * _C * _KCAP), _C * _KCAP)],
                ksc_v, sem),
            pltpu.async_copy(
                kid_hbm.at[pl.ds(_al8(b * _C * _KCAP), _C * _KCAP)],
                kid_v, sem),
            pltpu.async_copy(
                kbx_hbm.at[pl.ds(_al8(b * _C * 4 * _KCAP), _C * 4 * _KCAP)],
                kbx_v, sem),
            pltpu.async_copy(
                kcnt_hbm.at[pl.ds(_al8(b * _C * 16), _C * 16)], kcnt_v, sem),
        ]
        for cp in cps:
            cp.wait()

        def initf(k, _):
            osc_v[pl.ds(k * 16, 16)] = _bfull(jnp.float32(-1.0))
            olsc_v[pl.ds(k * 16, 16)] = _bfull(jnp.float32(-1.0))
            olab_v[pl.ds(k * 16, 16)] = _bfull(jnp.int32(-1))
            ollab_v[pl.ds(k * 16, 16)] = _bfull(jnp.int32(-1))
            sub_v[pl.ds(k * 16, 16)] = _bfull(jnp.int32(0))
            return 0
        lax.fori_loop(0, _KCAP // 16, initf, 0)

        def initb(k, _):
            obf_v[pl.ds(k * 16, 16)] = _bfull(jnp.float32(-1.0))
            return 0
        lax.fori_loop(0, (_KCAP * 4) // 16, initb, 0)

        def initi(k, _):
            @pl.when(k < 8)
            def _():
                idxa_v[pl.ds(k * 16, 16)] = _bfull(jnp.int32(0))
                idxb_v[pl.ds(k * 16, 16)] = _bfull(jnp.int32(0))

            @pl.when(k < 3)
            def _():
                idxc_v[pl.ds(k * 16, 16)] = _bfull(jnp.int32(0))

            return 0
        lax.fori_loop(0, 8, initi, 0)

        zeros = _bfull(jnp.int32(0))
        row1 = jnp.minimum(it16 + 16, jnp.int32(_C - 1))
        cnt0 = plsc.load_gather(kcnt_v, [it16 * 16])
        cnt1 = plsc.load_gather(kcnt_v, [row1 * 16])
        lane_ok1 = it16 < (_C - 16)

        s0i = plsc.load_gather(ksc_v, [it16 * _KCAP])
        s0i = jnp.where(cnt0 > 0, s0i, _NEG)
        s1i = plsc.load_gather(ksc_v, [row1 * _KCAP])
        s1i = jnp.where(lane_ok1 & (cnt1 > 0), s1i, _NEG)

        def mg_cond(carry):
            return carry[3]

        def mg_body(carry):
            r, h0, h1, _, s0, s1 = carry
            msp = _vmaxsplat(jnp.maximum(s0, s1))
            m_ok = msp[0] > _NEG

            def live(op):
                r, h0, h1, s0, s1 = op
                c0v = plsc.all_reduce_ffs(s0 == msp)
                c1v = plsc.all_reduce_ffs(s1 == msp)
                cbv = jnp.where(c0v < 16, c0v, c1v + 16)
                lanev = cbv & jnp.int32(15)
                in0 = cbv < 16
                hval = jnp.take(jnp.where(in0, h0, h1), lanev)
                sel0 = (it16 == lanev) & in0
                sel1 = (it16 == lanev) & (~in0)
                h0n = jnp.where(sel0, h0 + 1, h0)
                h1n = jnp.where(sel1, h1 + 1, h1)

                m0 = it16 == 0
                posv = _bfull(r, jnp.int32)
                plsc.store_scatter(osc_v, [posv], msp, mask=m0)
                plsc.store_scatter(olab_v, [posv], cbv, mask=m0)
                crd = jnp.minimum(it16, jnp.int32(3))
                bxv = plsc.load_gather(
                    kbx_v, [(cbv * 4 + crd) * _KCAP + hval])
                plsc.store_scatter(obf_v, [r * 4 + crd], bxv, mask=it16 < 4)
                fidx = plsc.load_gather(kid_v, [cbv * _KCAP + hval])
                gidx = fidx + b * _N
                ridx = gidx >> 3
                sub = gidx & jnp.int32(7)
                plsc.store_scatter(sub_v, [posv], sub, mask=m0)
                pa = _bfull(jnp.minimum(r, 127), jnp.int32)
                pb = _bfull(jnp.clip(r - 128, 0, 127), jnp.int32)
                pc = _bfull(jnp.clip(r - 256, 0, _KCAP - 257), jnp.int32)
                plsc.store_scatter(idxa_v, [pa], ridx, mask=m0 & (r < 128))
                plsc.store_scatter(idxb_v, [pb], ridx,
                                   mask=m0 & (r >= 128) & (r < 256))
                plsc.store_scatter(idxc_v, [pc], ridx, mask=m0 & (r >= 256))

                hn = hval + 1
                cntb = plsc.load_gather(kcnt_v, [cbv * 16])
                news = plsc.load_gather(
                    ksc_v, [cbv * _KCAP + jnp.minimum(hn, _KCAP - 1)])
                news = jnp.where(hn < cntb, news, _NEG)
                s0n = jnp.where(sel0, news, s0)
                s1n = jnp.where(sel1, news, s1)
                return r + 1, h0n, h1n, s0n, s1n

            r2, h0n, h1n, s0n, s1n = lax.cond(
                m_ok, live, lambda op: op, (r, h0, h1, s0, s1))
            return r2, h0n, h1n, m_ok & (r2 < _MAXD), s0n, s1n

        rf, _, _, _, _, _ = lax.while_loop(
            mg_cond, mg_body, (jnp.int32(0), zeros, zeros, True, s0i, s1i))

        g1 = pltpu.async_copy(lc_hbm.at[idxa_v], lcr_v.at[pl.ds(0, 128)], sem)
        g2 = pltpu.async_copy(
            lc_hbm.at[idxb_v], lcr_v.at[pl.ds(128, 128)], sem)
        g3 = pltpu.async_copy(
            lc_hbm.at[idxc_v], lcr_v.at[pl.ds(256, _KCAP - 256)], sem)
        g1.wait()
        g2.wait()
        g3.wait()

        def lcl(r):
            rv = _bfull(r, jnp.int32)
            sub16 = plsc.load_gather(sub_v, [rv])
            row = plsc.load_gather(lcr_v, [rv, sub16 * 16 + it16])
            lmv = _vmaxsplat(row)
            lav = plsc.all_reduce_ffs(row == lmv)
            m0 = it16 == 0
            plsc.store_scatter(olsc_v, [rv], lmv, mask=m0)
            plsc.store_scatter(ollab_v, [rv], lav, mask=m0)
        plsc.parallel_loop(0, rf, 1, unroll=2)(lcl)

        ops = [
            pltpu.async_copy(
                obf_v, obox_hbm.at[pl.ds(_al8(b * _KCAP * 4), _KCAP * 4)],
                sem),
            pltpu.async_copy(
                osc_v, osc_hbm.at[pl.ds(_al8(b * _KCAP), _KCAP)], sem),
            pltpu.async_copy(
                olab_v, olab_hbm.at[pl.ds(_al8(b * _KCAP), _KCAP)], sem),
            pltpu.async_copy(
                olsc_v, olsc_hbm.at[pl.ds(_al8(b * _KCAP), _KCAP)], sem),
            pltpu.async_copy(
                ollab_v, ollab_hbm.at[pl.ds(_al8(b * _KCAP), _KCAP)], sem),
        ]
        for op in ops:
            op.wait()


_phase2 = pl.kernel(
    _merge_body,
    out_type=(
        jax.ShapeDtypeStruct((_B * _KCAP * 4,), jnp.float32),
        jax.ShapeDtypeStruct((_B * _KCAP,), jnp.float32),
        jax.ShapeDtypeStruct((_B * _KCAP,), jnp.int32),
        jax.ShapeDtypeStruct((_B * _KCAP,), jnp.float32),
        jax.ShapeDtypeStruct((_B * _KCAP,), jnp.int32),
    ),
    mesh=_mesh,
    compiler_params=_params,
    scratch_types=[
        pltpu.VMEM((_C * _KCAP,), jnp.float32),
        pltpu.VMEM((_C * _KCAP,), jnp.int32),
        pltpu.VMEM((_C * 4 * _KCAP,), jnp.float32),
        pltpu.VMEM((_C * 16,), jnp.int32),
        pltpu.VMEM((_KCAP * 4,), jnp.float32),
        pltpu.VMEM((_KCAP,), jnp.float32),
        pltpu.VMEM((_KCAP,), jnp.int32),
        pltpu.VMEM((_KCAP,), jnp.float32),
        pltpu.VMEM((_KCAP,), jnp.int32),
        pltpu.VMEM((128,), jnp.int32),
        pltpu.VMEM((128,), jnp.int32),
        pltpu.VMEM((_KCAP - 256,), jnp.int32),
        pltpu.VMEM((_KCAP,), jnp.int32),
        pltpu.VMEM((_KCAP, _LCROW), jnp.float32),
        pltpu.SemaphoreType.DMA,
    ],
)


@jax.jit
def kernel(boxes, classification, l_classification):
    cls_t = jnp.transpose(classification, (0, 2, 1)).reshape(-1)
    bx_t = jnp.transpose(boxes, (0, 2, 1)).reshape(-1)
    lc = jnp.pad(
        l_classification, ((0, 0), (0, 0), (0, 16 - _LC)),
        constant_values=-np.inf,
    ).reshape(_B * _N * 16 // _LCROW, _LCROW)
    ksc, kid, kbx, kcnt = _phase1(cls_t, bx_t)
    obox, osc, olab, olsc, ollab = _phase2(ksc, kid, kbx, kcnt, lc)
    return (
        obox.reshape(_B, _KCAP, 4)[:, :_MAXD],
        osc.reshape(_B, _KCAP)[:, :_MAXD],
        olab.reshape(_B, _KCAP)[:, :_MAXD],
        olsc.reshape(_B, _KCAP)[:, :_MAXD],
        ollab.reshape(_B, _KCAP)[:, :_MAXD],
    )
